# Initial kernel scaffold; baseline (speedup 1.0000x reference)
import jax
import jax.numpy as jnp
from jax.experimental import pallas as pl


def kernel(x, edge_attr, params, edge_index, line_graph_edge_index, edge_index_batch):
    raise NotImplementedError("write your pallas kernel here")


# speedup vs baseline: 1.1858x; 1.1858x over previous
import functools

import jax
import jax.numpy as jnp
from jax.experimental import pallas as pl
from jax.experimental.pallas import tpu as pltpu

N_NODES = 10000
N_EDGES = 320000
N_LG = 640000
N_GRAPHS = 128
D = 128
N_ITER = 3
SND = 6 * D

ROW_BLOCK = 1000


def _stats_body(x_ref, o_ref):
    i = pl.program_id(0)

    @pl.when(i == 0)
    def _():
        o_ref[...] = jnp.zeros_like(o_ref)

    xb = x_ref[...]
    o_ref[0, :] += jnp.sum(xb, axis=0)
    o_ref[1, :] += jnp.sum(xb * xb, axis=0)


def _col_stats(x):
    n, c = x.shape
    return pl.pallas_call(
        _stats_body,
        grid=(n // ROW_BLOCK,),
        in_specs=[pl.BlockSpec((ROW_BLOCK, c), lambda i: (i, 0))],
        out_specs=pl.BlockSpec((2, c), lambda i: (0, 0)),
        out_shape=jax.ShapeDtypeStruct((2, c), jnp.float32),
    )(x)


def _apply_body(x_ref, s_ref, g_ref, b_ref, w_ref, c_ref, p_ref, o_ref, *, n, use_prelu):
    m = s_ref[0, :] / n
    v = s_ref[1, :] / n - m * m
    xn = (x_ref[...] - m[None, :]) * (g_ref[0, :] / jnp.sqrt(v + 1e-5))[None, :] + b_ref[0, :][None, :]
    if use_prelu:
        p = p_ref[0, 0]
        xn = jnp.where(xn >= 0, xn, p * xn)
    o_ref[...] = jnp.dot(xn, w_ref[...], preferred_element_type=jnp.float32) + c_ref[0, :][None, :]


def _bn_prelu_matmul(x, stats, g, b, w, c, p):
    n, cin = x.shape
    cout = w.shape[1]
    use_prelu = p is not None
    if p is None:
        p_arr = jnp.zeros((1, 1), jnp.float32)
    else:
        p_arr = jnp.asarray(p, jnp.float32).reshape(1, 1)
    body = functools.partial(_apply_body, n=float(n), use_prelu=use_prelu)
    return pl.pallas_call(
        body,
        grid=(n // ROW_BLOCK,),
        in_specs=[
            pl.BlockSpec((ROW_BLOCK, cin), lambda i: (i, 0)),
            pl.BlockSpec((2, cin), lambda i: (0, 0)),
            pl.BlockSpec((1, cin), lambda i: (0, 0)),
            pl.BlockSpec((1, cin), lambda i: (0, 0)),
            pl.BlockSpec((cin, cout), lambda i: (0, 0)),
            pl.BlockSpec((1, cout), lambda i: (0, 0)),
            pl.BlockSpec((1, 1), lambda i: (0, 0)),
        ],
        out_specs=pl.BlockSpec((ROW_BLOCK, cout), lambda i: (i, 0)),
        out_shape=jax.ShapeDtypeStruct((n, cout), jnp.float32),
    )(x, stats, g.reshape(1, -1), b.reshape(1, -1), w, c.reshape(1, -1), p_arr)


def _stage(x, p, idx, use_prelu):
    i = str(idx)
    stats = _col_stats(x)
    pr = p['p' + i] if use_prelu else None
    return _bn_prelu_matmul(x, stats, p['g' + i], p['b' + i], p['W' + i], p['c' + i], pr)


def _linear_block_pallas(x, p):
    x1 = _stage(x, p, 1, False)
    x2 = _stage(x1, p, 2, True)
    x3 = _stage(x2, p, 3, True)
    xm = (x3 + x1) / 2.0
    x4 = _stage(xm, p, 4, True)
    xm2 = (x4 + xm) / 2.0
    return _stage(xm2, p, 5, True)


def kernel(x, edge_attr, params, edge_index, line_graph_edge_index, edge_index_batch):
    lg = line_graph_edge_index
    batch = edge_index_batch
    eu = x @ params['Wu']
    ev = x @ params['Wv']
    euv = edge_attr @ params['We']
    ea = (eu[edge_index[0]] + ev[edge_index[1]] + euv) / 3.0
    out = ea
    out_list = []
    gout_list = []
    for _ in range(N_ITER):
        agg = jax.ops.segment_sum(out[lg[0]], lg[1], num_segments=N_EDGES)
        out = ea + agg
        conv_agg = jax.ops.segment_sum(out[lg[0]], lg[1], num_segments=N_EDGES)
        xc = conv_agg @ params['Wrel'] + params['crel'] + out @ params['Wroot']
        smax = jax.ops.segment_max(xc, batch, num_segments=N_GRAPHS)
        ex = jnp.exp(xc - smax[batch])
        den = jax.ops.segment_sum(ex, batch, num_segments=N_GRAPHS)
        scores = ex / den[batch]
        gx = jax.ops.segment_sum(out * scores, batch, num_segments=N_GRAPHS)
        out_list.append(out)
        gout_list.append(jnp.tanh(gx @ params['Wgout'] + params['cgout']))
    gout_all = jnp.stack(gout_list, axis=-1)
    out_all = jnp.stack(out_list, axis=-1)
    sc = jnp.sum(gout_all * params['a'], axis=1, keepdims=True) + params['a_bias']
    sc = jax.nn.softmax(sc, axis=-1)
    sc_e = sc[batch]
    out = jnp.sum(out_all * sc_e, axis=-1)
    node_agg = jax.ops.segment_sum(out, edge_index[1], num_segments=N_NODES)
    h = x + node_agg
    return _linear_block_pallas(h, params)


# speedup vs baseline: 1.4444x; 1.2181x over previous
import dataclasses
import functools

import jax
import jax.numpy as jnp
from jax import lax
from jax.experimental import pallas as pl
from jax.experimental.pallas import tpu as pltpu
from jax.experimental.pallas import tpu_sc as plsc

N_NODES = 10000
N_EDGES = 320000
N_LG = 640000
N_GRAPHS = 128
D = 128
N_ITER = 3
SND = 6 * D

ROW_BLOCK = 1000


_E = 6400
_CHUNK = 128
_NSUB = 16


def _ptr_pad(n):
    return (n + 15) // 16 * 16


def _vext(vref, idx):
    base = pl.multiple_of((idx >> 4) << 4, 8)
    grp = vref[pl.ds(base, 16)]
    msk = lax.broadcasted_iota(jnp.int32, (16,), 0) == (idx & 15)
    return jnp.sum(jnp.where(msk, grp, 0), axis=0)


def _seg_body(T, E, src_hbm, base_hbm, ssrc_hbm, sdst_hbm, tptr_hbm, out_hbm,
              tptr_v, idx_v, dst_v, ldst_v, rows_v, acc_sh):
    c = lax.axis_index("c")
    s = lax.axis_index("s")
    ER = E // _NSUB
    pltpu.sync_copy(tptr_hbm, tptr_v)
    n_my_tiles = (T - c + 1) // 2

    def tile_body(i, carry):
        t = c + 2 * i
        tbase = t * E
        pltpu.sync_copy(base_hbm.at[pl.ds(tbase + s * ER, ER)],
                        acc_sh.at[pl.ds(s * ER, ER)])
        plsc.subcore_barrier()
        a = _vext(tptr_v, t)
        b = _vext(tptr_v, t + 1)
        lo = (a >> 3) << 3
        w8 = ((b + 7) >> 3) - (a >> 3)
        p0 = lo + ((w8 * s) >> 4) * 8
        p1 = lo + ((w8 * (s + 1)) >> 4) * 8
        nch = (p1 - p0 + _CHUNK - 1) // _CHUNK

        def chunk_body(j, carry2):
            off = pl.multiple_of(p0 + j * _CHUNK, 8)
            pltpu.sync_copy(ssrc_hbm.at[pl.ds(off, _CHUNK)], idx_v)
            pltpu.sync_copy(sdst_hbm.at[pl.ds(off, _CHUNK)], dst_v)
            for v in range(_CHUNK // 16):
                dv = dst_v[pl.ds(v * 16, 16)]
                ld = dv - tbase
                slot = off + v * 16 + lax.broadcasted_iota(jnp.int32, (16,), 0)
                ok = (ld >= 0) & (ld < E) & (slot < p1)
                ldst_v[pl.ds(v * 16, 16)] = jnp.where(ok, ld, E)
            pltpu.sync_copy(src_hbm.at[idx_v], rows_v)
            pltpu.sync_copy(rows_v, acc_sh.at[ldst_v], add=True)
            return carry2

        lax.fori_loop(0, nch, chunk_body, 0)
        plsc.subcore_barrier()
        pltpu.sync_copy(acc_sh.at[pl.ds(s * ER, ER)],
                        out_hbm.at[pl.ds(tbase + s * ER, ER)])
        plsc.subcore_barrier()
        return carry

    lax.fori_loop(0, n_my_tiles, tile_body, 0)


def _sc_segsum(src, base, ssrc_pad, sdst_pad, tptr, n_out, E):
    T = n_out // E
    mesh = plsc.VectorSubcoreMesh(core_axis_name="c", subcore_axis_name="s")
    body = functools.partial(_seg_body, T, E)
    cp = pltpu.CompilerParams()
    if "needs_layout_passes" in pltpu.CompilerParams.__dataclass_fields__:
        cp = dataclasses.replace(cp, needs_layout_passes=False)
    f = pl.kernel(
        body,
        out_type=jax.ShapeDtypeStruct((n_out, D), jnp.float32),
        mesh=mesh,
        compiler_params=cp,
        scratch_types=[
            pltpu.VMEM((_ptr_pad(T + 1),), jnp.int32),
            pltpu.VMEM((_CHUNK,), jnp.int32),
            pltpu.VMEM((_CHUNK,), jnp.int32),
            pltpu.VMEM((_CHUNK,), jnp.int32),
            pltpu.VMEM((_CHUNK, D), jnp.float32),
            pltpu.VMEM_SHARED((E + 8, D), jnp.float32),
        ],
    )
    return f(src, base, ssrc_pad, sdst_pad, tptr)


def _sort_lg(lg_dst, lg_src, n_out, E):
    sdst, ssrc = lax.sort((lg_dst, lg_src), dimension=0, num_keys=1)
    T = n_out // E
    tptr = jnp.searchsorted(sdst, jnp.arange(T + 1, dtype=jnp.int32) * E).astype(jnp.int32)
    tptr = jnp.concatenate([tptr, jnp.full((_ptr_pad(T + 1) - (T + 1),), sdst.shape[0], jnp.int32)])
    ssrc_pad = jnp.concatenate([ssrc, jnp.zeros((_CHUNK,), jnp.int32)])
    sdst_pad = jnp.concatenate([sdst, jnp.full((_CHUNK,), n_out, jnp.int32)])
    return ssrc_pad, sdst_pad, tptr


def _stats_body(x_ref, o_ref):
    i = pl.program_id(0)

    @pl.when(i == 0)
    def _():
        o_ref[...] = jnp.zeros_like(o_ref)

    xb = x_ref[...]
    o_ref[0, :] += jnp.sum(xb, axis=0)
    o_ref[1, :] += jnp.sum(xb * xb, axis=0)


def _col_stats(x):
    n, c = x.shape
    return pl.pallas_call(
        _stats_body,
        grid=(n // ROW_BLOCK,),
        in_specs=[pl.BlockSpec((ROW_BLOCK, c), lambda i: (i, 0))],
        out_specs=pl.BlockSpec((2, c), lambda i: (0, 0)),
        out_shape=jax.ShapeDtypeStruct((2, c), jnp.float32),
    )(x)


def _apply_body(x_ref, s_ref, g_ref, b_ref, w_ref, c_ref, p_ref, o_ref, *, n, use_prelu):
    m = s_ref[0, :] / n
    v = s_ref[1, :] / n - m * m
    xn = (x_ref[...] - m[None, :]) * (g_ref[0, :] / jnp.sqrt(v + 1e-5))[None, :] + b_ref[0, :][None, :]
    if use_prelu:
        p = p_ref[0, 0]
        xn = jnp.where(xn >= 0, xn, p * xn)
    o_ref[...] = jnp.dot(xn, w_ref[...], preferred_element_type=jnp.float32) + c_ref[0, :][None, :]


def _bn_prelu_matmul(x, stats, g, b, w, c, p):
    n, cin = x.shape
    cout = w.shape[1]
    use_prelu = p is not None
    if p is None:
        p_arr = jnp.zeros((1, 1), jnp.float32)
    else:
        p_arr = jnp.asarray(p, jnp.float32).reshape(1, 1)
    body = functools.partial(_apply_body, n=float(n), use_prelu=use_prelu)
    return pl.pallas_call(
        body,
        grid=(n // ROW_BLOCK,),
        in_specs=[
            pl.BlockSpec((ROW_BLOCK, cin), lambda i: (i, 0)),
            pl.BlockSpec((2, cin), lambda i: (0, 0)),
            pl.BlockSpec((1, cin), lambda i: (0, 0)),
            pl.BlockSpec((1, cin), lambda i: (0, 0)),
            pl.BlockSpec((cin, cout), lambda i: (0, 0)),
            pl.BlockSpec((1, cout), lambda i: (0, 0)),
            pl.BlockSpec((1, 1), lambda i: (0, 0)),
        ],
        out_specs=pl.BlockSpec((ROW_BLOCK, cout), lambda i: (i, 0)),
        out_shape=jax.ShapeDtypeStruct((n, cout), jnp.float32),
    )(x, stats, g.reshape(1, -1), b.reshape(1, -1), w, c.reshape(1, -1), p_arr)


def _stage(x, p, idx, use_prelu):
    i = str(idx)
    stats = _col_stats(x)
    pr = p['p' + i] if use_prelu else None
    return _bn_prelu_matmul(x, stats, p['g' + i], p['b' + i], p['W' + i], p['c' + i], pr)


def _linear_block_pallas(x, p):
    x1 = _stage(x, p, 1, False)
    x2 = _stage(x1, p, 2, True)
    x3 = _stage(x2, p, 3, True)
    xm = (x3 + x1) / 2.0
    x4 = _stage(xm, p, 4, True)
    xm2 = (x4 + xm) / 2.0
    return _stage(xm2, p, 5, True)


def kernel(x, edge_attr, params, edge_index, line_graph_edge_index, edge_index_batch):
    lg = line_graph_edge_index
    batch = edge_index_batch
    eu = x @ params['Wu']
    ev = x @ params['Wv']
    euv = edge_attr @ params['We']
    ea = (eu[edge_index[0]] + ev[edge_index[1]] + euv) / 3.0
    ssrc_pad, sdst_pad, tptr = _sort_lg(lg[1], lg[0], N_EDGES, _E)
    zeros_base = jnp.zeros((N_EDGES, D), jnp.float32)
    out = ea
    out_list = []
    gout_list = []
    for _ in range(N_ITER):
        out = _sc_segsum(out, ea, ssrc_pad, sdst_pad, tptr, N_EDGES, _E)
        conv_agg = _sc_segsum(out, zeros_base, ssrc_pad, sdst_pad, tptr, N_EDGES, _E)
        xc = conv_agg @ params['Wrel'] + params['crel'] + out @ params['Wroot']
        smax = jax.ops.segment_max(xc, batch, num_segments=N_GRAPHS)
        ex = jnp.exp(xc - smax[batch])
        den = jax.ops.segment_sum(ex, batch, num_segments=N_GRAPHS)
        scores = ex / den[batch]
        gx = jax.ops.segment_sum(out * scores, batch, num_segments=N_GRAPHS)
        out_list.append(out)
        gout_list.append(jnp.tanh(gx @ params['Wgout'] + params['cgout']))
    gout_all = jnp.stack(gout_list, axis=-1)
    out_all = jnp.stack(out_list, axis=-1)
    sc = jnp.sum(gout_all * params['a'], axis=1, keepdims=True) + params['a_bias']
    sc = jax.nn.softmax(sc, axis=-1)
    sc_e = sc[batch]
    out = jnp.sum(out_all * sc_e, axis=-1)
    node_agg = jax.ops.segment_sum(out, edge_index[1], num_segments=N_NODES)
    h = x + node_agg
    return _linear_block_pallas(h, params)


# speedup vs baseline: 2.6968x; 1.8671x over previous
import dataclasses
import functools

import jax
import jax.numpy as jnp
from jax import lax
from jax.experimental import pallas as pl
from jax.experimental.pallas import tpu as pltpu
from jax.experimental.pallas import tpu_sc as plsc

N_NODES = 10000
N_EDGES = 320000
N_LG = 640000
N_GRAPHS = 128
D = 128
N_ITER = 3
SND = 6 * D

ROW_BLOCK = 1000


_E = 6400
_CHUNK = 128
_NSUB = 16


def _ptr_pad(n):
    return (n + 15) // 16 * 16


def _vext(vref, idx):
    base = pl.multiple_of((idx >> 4) << 4, 8)
    grp = vref[pl.ds(base, 16)]
    msk = lax.broadcasted_iota(jnp.int32, (16,), 0) == (idx & 15)
    return jnp.sum(jnp.where(msk, grp, 0), axis=0)


def _seg_body(T, E, src_hbm, base_hbm, ssrc_hbm, sdst_hbm, tptr_hbm, out_hbm,
              tptr_v, idx_v, dst_v, ldst_v, rows_v, acc_sh):
    c = lax.axis_index("c")
    s = lax.axis_index("s")
    ER = E // _NSUB
    pltpu.sync_copy(tptr_hbm, tptr_v)
    n_my_tiles = (T - c + 1) // 2

    def tile_body(i, carry):
        t = c + 2 * i
        tbase = t * E
        pltpu.sync_copy(base_hbm.at[pl.ds(tbase + s * ER, ER)],
                        acc_sh.at[pl.ds(s * ER, ER)])
        plsc.subcore_barrier()
        a = _vext(tptr_v, t)
        b = _vext(tptr_v, t + 1)
        lo = (a >> 3) << 3
        w8 = ((b + 7) >> 3) - (a >> 3)
        p0 = lo + ((w8 * s) >> 4) * 8
        p1 = lo + ((w8 * (s + 1)) >> 4) * 8
        nch = (p1 - p0 + _CHUNK - 1) // _CHUNK

        def chunk_body(j, carry2):
            off = pl.multiple_of(p0 + j * _CHUNK, 8)
            pltpu.sync_copy(ssrc_hbm.at[pl.ds(off, _CHUNK)], idx_v)
            pltpu.sync_copy(sdst_hbm.at[pl.ds(off, _CHUNK)], dst_v)
            for v in range(_CHUNK // 16):
                dv = dst_v[pl.ds(v * 16, 16)]
                ld = dv - tbase
                slot = off + v * 16 + lax.broadcasted_iota(jnp.int32, (16,), 0)
                ok = (ld >= 0) & (ld < E) & (slot < p1)
                ldst_v[pl.ds(v * 16, 16)] = jnp.where(ok, ld, E)
            pltpu.sync_copy(src_hbm.at[idx_v], rows_v)
            pltpu.sync_copy(rows_v, acc_sh.at[ldst_v], add=True)
            return carry2

        lax.fori_loop(0, nch, chunk_body, 0)
        plsc.subcore_barrier()
        pltpu.sync_copy(acc_sh.at[pl.ds(s * ER, ER)],
                        out_hbm.at[pl.ds(tbase + s * ER, ER)])
        plsc.subcore_barrier()
        return carry

    lax.fori_loop(0, n_my_tiles, tile_body, 0)


def _sc_segsum(src, base, ssrc_pad, sdst_pad, tptr, n_out, E):
    T = n_out // E
    mesh = plsc.VectorSubcoreMesh(core_axis_name="c", subcore_axis_name="s")
    body = functools.partial(_seg_body, T, E)
    cp = pltpu.CompilerParams()
    if "needs_layout_passes" in pltpu.CompilerParams.__dataclass_fields__:
        cp = dataclasses.replace(cp, needs_layout_passes=False)
    f = pl.kernel(
        body,
        out_type=jax.ShapeDtypeStruct((n_out, D), jnp.float32),
        mesh=mesh,
        compiler_params=cp,
        scratch_types=[
            pltpu.VMEM((_ptr_pad(T + 1),), jnp.int32),
            pltpu.VMEM((_CHUNK,), jnp.int32),
            pltpu.VMEM((_CHUNK,), jnp.int32),
            pltpu.VMEM((_CHUNK,), jnp.int32),
            pltpu.VMEM((_CHUNK, D), jnp.float32),
            pltpu.VMEM_SHARED((E + 8, D), jnp.float32),
        ],
    )
    return f(src, base, ssrc_pad, sdst_pad, tptr)


def _sort_lg(lg_dst, lg_src, n_out, E):
    sdst, ssrc = lax.sort((lg_dst, lg_src), dimension=0, num_keys=1)
    T = n_out // E
    tptr = jnp.searchsorted(sdst, jnp.arange(T + 1, dtype=jnp.int32) * E).astype(jnp.int32)
    tptr = jnp.concatenate([tptr, jnp.full((_ptr_pad(T + 1) - (T + 1),), sdst.shape[0], jnp.int32)])
    ssrc_pad = jnp.concatenate([ssrc, jnp.zeros((_CHUNK,), jnp.int32)])
    sdst_pad = jnp.concatenate([sdst, jnp.full((_CHUNK,), n_out, jnp.int32)])
    return ssrc_pad, sdst_pad, tptr


def _stats_body(x_ref, o_ref):
    i = pl.program_id(0)

    @pl.when(i == 0)
    def _():
        o_ref[...] = jnp.zeros_like(o_ref)

    xb = x_ref[...]
    o_ref[0, :] += jnp.sum(xb, axis=0)
    o_ref[1, :] += jnp.sum(xb * xb, axis=0)


def _col_stats(x):
    n, c = x.shape
    return pl.pallas_call(
        _stats_body,
        grid=(n // ROW_BLOCK,),
        in_specs=[pl.BlockSpec((ROW_BLOCK, c), lambda i: (i, 0))],
        out_specs=pl.BlockSpec((2, c), lambda i: (0, 0)),
        out_shape=jax.ShapeDtypeStruct((2, c), jnp.float32),
    )(x)


def _apply_body(x_ref, s_ref, g_ref, b_ref, w_ref, c_ref, p_ref, o_ref, *, n, use_prelu):
    m = s_ref[0, :] / n
    v = s_ref[1, :] / n - m * m
    xn = (x_ref[...] - m[None, :]) * (g_ref[0, :] / jnp.sqrt(v + 1e-5))[None, :] + b_ref[0, :][None, :]
    if use_prelu:
        p = p_ref[0, 0]
        xn = jnp.where(xn >= 0, xn, p * xn)
    o_ref[...] = jnp.dot(xn, w_ref[...], preferred_element_type=jnp.float32) + c_ref[0, :][None, :]


def _bn_prelu_matmul(x, stats, g, b, w, c, p):
    n, cin = x.shape
    cout = w.shape[1]
    use_prelu = p is not None
    if p is None:
        p_arr = jnp.zeros((1, 1), jnp.float32)
    else:
        p_arr = jnp.asarray(p, jnp.float32).reshape(1, 1)
    body = functools.partial(_apply_body, n=float(n), use_prelu=use_prelu)
    return pl.pallas_call(
        body,
        grid=(n // ROW_BLOCK,),
        in_specs=[
            pl.BlockSpec((ROW_BLOCK, cin), lambda i: (i, 0)),
            pl.BlockSpec((2, cin), lambda i: (0, 0)),
            pl.BlockSpec((1, cin), lambda i: (0, 0)),
            pl.BlockSpec((1, cin), lambda i: (0, 0)),
            pl.BlockSpec((cin, cout), lambda i: (0, 0)),
            pl.BlockSpec((1, cout), lambda i: (0, 0)),
            pl.BlockSpec((1, 1), lambda i: (0, 0)),
        ],
        out_specs=pl.BlockSpec((ROW_BLOCK, cout), lambda i: (i, 0)),
        out_shape=jax.ShapeDtypeStruct((n, cout), jnp.float32),
    )(x, stats, g.reshape(1, -1), b.reshape(1, -1), w, c.reshape(1, -1), p_arr)


_PBLK = 2000


def _smax_body(xc_ref, b_ref, o_ref):
    i = pl.program_id(0)

    @pl.when(i == 0)
    def _():
        o_ref[...] = jnp.full_like(o_ref, -1e30)

    gid = lax.broadcasted_iota(jnp.int32, (1, N_GRAPHS), 1)
    oh = b_ref[...] == gid
    vals = jnp.where(oh, xc_ref[...], -1e30)
    o_ref[...] = jnp.maximum(o_ref[...], jnp.max(vals, axis=0, keepdims=True))


def _pool_body(xc_ref, b_ref, out_ref, smax_ref, den_ref, gx_ref):
    i = pl.program_id(0)

    @pl.when(i == 0)
    def _():
        den_ref[...] = jnp.zeros_like(den_ref)
        gx_ref[...] = jnp.zeros_like(gx_ref)

    gid = lax.broadcasted_iota(jnp.int32, (1, N_GRAPHS), 1)
    oh = b_ref[...] == gid
    smax_sel = jnp.max(jnp.where(oh, smax_ref[...], -1e30), axis=1, keepdims=True)
    ex = jnp.exp(xc_ref[...] - smax_sel)
    exoh = oh.astype(jnp.float32) * ex
    den_ref[...] += jnp.sum(exoh, axis=0, keepdims=True)
    gx_ref[...] += lax.dot_general(exoh, out_ref[...], (((0,), (0,)), ((), ())),
                                   preferred_element_type=jnp.float32)


def _pool(xc, batch_col, out):
    nb = N_EDGES // _PBLK
    colspec = pl.BlockSpec((_PBLK, 1), lambda i: (i, 0))
    smax = pl.pallas_call(
        _smax_body,
        grid=(nb,),
        in_specs=[colspec, colspec],
        out_specs=pl.BlockSpec((1, N_GRAPHS), lambda i: (0, 0)),
        out_shape=jax.ShapeDtypeStruct((1, N_GRAPHS), jnp.float32),
    )(xc, batch_col)
    den, gxr = pl.pallas_call(
        _pool_body,
        grid=(nb,),
        in_specs=[
            colspec,
            colspec,
            pl.BlockSpec((_PBLK, D), lambda i: (i, 0)),
            pl.BlockSpec((1, N_GRAPHS), lambda i: (0, 0)),
        ],
        out_specs=[
            pl.BlockSpec((1, N_GRAPHS), lambda i: (0, 0)),
            pl.BlockSpec((N_GRAPHS, D), lambda i: (0, 0)),
        ],
        out_shape=[
            jax.ShapeDtypeStruct((1, N_GRAPHS), jnp.float32),
            jax.ShapeDtypeStruct((N_GRAPHS, D), jnp.float32),
        ],
    )(xc, batch_col, out, smax)
    return gxr / jnp.maximum(den, 1e-30).T


def _stage(x, p, idx, use_prelu):
    i = str(idx)
    stats = _col_stats(x)
    pr = p['p' + i] if use_prelu else None
    return _bn_prelu_matmul(x, stats, p['g' + i], p['b' + i], p['W' + i], p['c' + i], pr)


def _linear_block_pallas(x, p):
    x1 = _stage(x, p, 1, False)
    x2 = _stage(x1, p, 2, True)
    x3 = _stage(x2, p, 3, True)
    xm = (x3 + x1) / 2.0
    x4 = _stage(xm, p, 4, True)
    xm2 = (x4 + xm) / 2.0
    return _stage(xm2, p, 5, True)


def kernel(x, edge_attr, params, edge_index, line_graph_edge_index, edge_index_batch):
    lg = line_graph_edge_index
    batch = edge_index_batch
    eu = x @ params['Wu']
    ev = x @ params['Wv']
    euv = edge_attr @ params['We']
    ea = (eu[edge_index[0]] + ev[edge_index[1]] + euv) / 3.0
    ssrc_pad, sdst_pad, tptr = _sort_lg(lg[1], lg[0], N_EDGES, _E)
    zeros_base = jnp.zeros((N_EDGES, D), jnp.float32)
    batch_col = batch.reshape(-1, 1)
    out = ea
    out_list = []
    gout_list = []
    for _ in range(N_ITER):
        out = _sc_segsum(out, ea, ssrc_pad, sdst_pad, tptr, N_EDGES, _E)
        conv_agg = _sc_segsum(out, zeros_base, ssrc_pad, sdst_pad, tptr, N_EDGES, _E)
        xc = conv_agg @ params['Wrel'] + params['crel'] + out @ params['Wroot']
        gx = _pool(xc, batch_col, out)
        out_list.append(out)
        gout_list.append(jnp.tanh(gx @ params['Wgout'] + params['cgout']))
    gout_all = jnp.stack(gout_list, axis=-1)
    out_all = jnp.stack(out_list, axis=-1)
    sc = jnp.sum(gout_all * params['a'], axis=1, keepdims=True) + params['a_bias']
    sc = jax.nn.softmax(sc, axis=-1)
    sc_e = sc[batch]
    out = jnp.sum(out_all * sc_e, axis=-1)
    node_agg = jax.ops.segment_sum(out, edge_index[1], num_segments=N_NODES)
    h = x + node_agg
    return _linear_block_pallas(h, params)


# speedup vs baseline: 3.0137x; 1.1175x over previous
import dataclasses
import functools

import jax
import jax.numpy as jnp
from jax import lax
from jax.experimental import pallas as pl
from jax.experimental.pallas import tpu as pltpu
from jax.experimental.pallas import tpu_sc as plsc

N_NODES = 10000
N_EDGES = 320000
N_LG = 640000
N_GRAPHS = 128
D = 128
N_ITER = 3
SND = 6 * D

ROW_BLOCK = 1000


_E = 6400
_CHUNK = 128
_NSUB = 16


def _ptr_pad(n):
    return (n + 15) // 16 * 16


def _vext(vref, idx):
    base = pl.multiple_of((idx >> 4) << 4, 8)
    grp = vref[pl.ds(base, 16)]
    msk = lax.broadcasted_iota(jnp.int32, (16,), 0) == (idx & 15)
    return jnp.sum(jnp.where(msk, grp, 0), axis=0)


_IBLK = 1024
_NCH = _IBLK // _CHUNK


def _zrows(ER):
    for d in range(min(128, ER), 0, -1):
        if ER % d == 0:
            return d


def _seg_body(T, E, init_zero, src_hbm, base_hbm, ssrc_hbm, sdst_hbm, tptr_hbm,
              out_hbm, tptr_v, isrc_v, idst_v, ldstA, ldstB, rowsA, rowsB,
              zbuf, acc_sh, semA, semB):
    c = lax.axis_index("c")
    s = lax.axis_index("s")
    ER = E // _NSUB
    ZR = _zrows(ER)
    pltpu.sync_copy(tptr_hbm, tptr_v)
    if init_zero:
        pltpu.sync_copy(base_hbm, zbuf)
    n_my_tiles = (T - c + 1) // 2
    iota16 = lax.broadcasted_iota(jnp.int32, (16,), 0)

    def tile_body(i, carry):
        t = c + 2 * i
        tbase = t * E
        if init_zero:
            for k in range(ER // ZR):
                pltpu.sync_copy(zbuf, acc_sh.at[pl.ds(s * ER + k * ZR, ZR)])
        else:
            pltpu.sync_copy(base_hbm.at[pl.ds(tbase + s * ER, ER)],
                            acc_sh.at[pl.ds(s * ER, ER)])
        plsc.subcore_barrier()
        a = _vext(tptr_v, t)
        b = _vext(tptr_v, t + 1)
        lo = (a >> 3) << 3
        w8 = ((b + 7) >> 3) - (a >> 3)
        p0 = lo + ((w8 * s) >> 4) * 8
        p1 = lo + ((w8 * (s + 1)) >> 4) * 8
        nblk = (p1 - p0 + _IBLK - 1) // _IBLK

        rows = (rowsA, rowsB)
        sems = (semA, semB)
        lds = (ldstA, ldstB)

        def blk_body(bi, carry2):
            boff = pl.multiple_of(p0 + bi * _IBLK, 8)
            pltpu.sync_copy(ssrc_hbm.at[pl.ds(boff, _IBLK)], isrc_v)
            pltpu.sync_copy(sdst_hbm.at[pl.ds(boff, _IBLK)], idst_v)

            def compute_ldst(ci, lref):
                for v in range(_CHUNK // 16):
                    o = ci * _CHUNK + v * 16
                    dv = idst_v[pl.ds(o, 16)]
                    ld = dv - tbase
                    slot = boff + o + iota16
                    ok = (ld >= 0) & (ld < E) & (slot < p1)
                    lref[pl.ds(v * 16, 16)] = jnp.where(ok, ld, E)

            def finish(ci, cp):
                lref = lds[ci % 2]
                compute_ldst(ci, lref)
                cp.wait()
                pltpu.sync_copy(rows[ci % 2], acc_sh.at[lref], add=True)

            cps = [None, None]
            for ci in range(_NCH):
                cps[ci % 2] = pltpu.async_copy(
                    src_hbm.at[isrc_v.at[pl.ds(ci * _CHUNK, _CHUNK)]],
                    rows[ci % 2], sems[ci % 2])
                if ci >= 1:
                    finish(ci - 1, cps[(ci - 1) % 2])
            finish(_NCH - 1, cps[(_NCH - 1) % 2])
            return carry2

        lax.fori_loop(0, nblk, blk_body, 0)
        plsc.subcore_barrier()
        pltpu.sync_copy(acc_sh.at[pl.ds(s * ER, ER)],
                        out_hbm.at[pl.ds(tbase + s * ER, ER)])
        plsc.subcore_barrier()
        return carry

    lax.fori_loop(0, n_my_tiles, tile_body, 0)


def _sc_segsum(src, base, ssrc_pad, sdst_pad, tptr, n_out, E, init_zero=False):
    T = n_out // E
    mesh = plsc.VectorSubcoreMesh(core_axis_name="c", subcore_axis_name="s")
    body = functools.partial(_seg_body, T, E, init_zero)
    cp = pltpu.CompilerParams()
    if "needs_layout_passes" in pltpu.CompilerParams.__dataclass_fields__:
        cp = dataclasses.replace(cp, needs_layout_passes=False)
    f = pl.kernel(
        body,
        out_type=jax.ShapeDtypeStruct((n_out, D), jnp.float32),
        mesh=mesh,
        compiler_params=cp,
        scratch_types=[
            pltpu.VMEM((_ptr_pad(T + 1),), jnp.int32),
            pltpu.VMEM((_IBLK,), jnp.int32),
            pltpu.VMEM((_IBLK,), jnp.int32),
            pltpu.VMEM((_CHUNK,), jnp.int32),
            pltpu.VMEM((_CHUNK,), jnp.int32),
            pltpu.VMEM((_CHUNK, D), jnp.float32),
            pltpu.VMEM((_CHUNK, D), jnp.float32),
            pltpu.VMEM((_zrows(E // _NSUB), D), jnp.float32),
            pltpu.VMEM_SHARED((E + 8, D), jnp.float32),
            pltpu.SemaphoreType.DMA,
            pltpu.SemaphoreType.DMA,
        ],
    )
    return f(src, base, ssrc_pad, sdst_pad, tptr)


def _sort_lg(lg_dst, lg_src, n_out, E):
    sdst, ssrc = lax.sort((lg_dst, lg_src), dimension=0, num_keys=1)
    T = n_out // E
    tptr = jnp.searchsorted(sdst, jnp.arange(T + 1, dtype=jnp.int32) * E).astype(jnp.int32)
    tptr = jnp.concatenate([tptr, jnp.full((_ptr_pad(T + 1) - (T + 1),), sdst.shape[0], jnp.int32)])
    ssrc_pad = jnp.concatenate([ssrc, jnp.zeros((2 * _IBLK,), jnp.int32)])
    sdst_pad = jnp.concatenate([sdst, jnp.full((2 * _IBLK,), n_out, jnp.int32)])
    return ssrc_pad, sdst_pad, tptr


def _stats_body(x_ref, o_ref):
    i = pl.program_id(0)

    @pl.when(i == 0)
    def _():
        o_ref[...] = jnp.zeros_like(o_ref)

    xb = x_ref[...]
    o_ref[0, :] += jnp.sum(xb, axis=0)
    o_ref[1, :] += jnp.sum(xb * xb, axis=0)


def _col_stats(x):
    n, c = x.shape
    return pl.pallas_call(
        _stats_body,
        grid=(n // ROW_BLOCK,),
        in_specs=[pl.BlockSpec((ROW_BLOCK, c), lambda i: (i, 0))],
        out_specs=pl.BlockSpec((2, c), lambda i: (0, 0)),
        out_shape=jax.ShapeDtypeStruct((2, c), jnp.float32),
    )(x)


def _apply_body(x_ref, s_ref, g_ref, b_ref, w_ref, c_ref, p_ref, o_ref, *, n, use_prelu):
    m = s_ref[0, :] / n
    v = s_ref[1, :] / n - m * m
    xn = (x_ref[...] - m[None, :]) * (g_ref[0, :] / jnp.sqrt(v + 1e-5))[None, :] + b_ref[0, :][None, :]
    if use_prelu:
        p = p_ref[0, 0]
        xn = jnp.where(xn >= 0, xn, p * xn)
    o_ref[...] = jnp.dot(xn, w_ref[...], preferred_element_type=jnp.float32) + c_ref[0, :][None, :]


def _bn_prelu_matmul(x, stats, g, b, w, c, p):
    n, cin = x.shape
    cout = w.shape[1]
    use_prelu = p is not None
    if p is None:
        p_arr = jnp.zeros((1, 1), jnp.float32)
    else:
        p_arr = jnp.asarray(p, jnp.float32).reshape(1, 1)
    body = functools.partial(_apply_body, n=float(n), use_prelu=use_prelu)
    return pl.pallas_call(
        body,
        grid=(n // ROW_BLOCK,),
        in_specs=[
            pl.BlockSpec((ROW_BLOCK, cin), lambda i: (i, 0)),
            pl.BlockSpec((2, cin), lambda i: (0, 0)),
            pl.BlockSpec((1, cin), lambda i: (0, 0)),
            pl.BlockSpec((1, cin), lambda i: (0, 0)),
            pl.BlockSpec((cin, cout), lambda i: (0, 0)),
            pl.BlockSpec((1, cout), lambda i: (0, 0)),
            pl.BlockSpec((1, 1), lambda i: (0, 0)),
        ],
        out_specs=pl.BlockSpec((ROW_BLOCK, cout), lambda i: (i, 0)),
        out_shape=jax.ShapeDtypeStruct((n, cout), jnp.float32),
    )(x, stats, g.reshape(1, -1), b.reshape(1, -1), w, c.reshape(1, -1), p_arr)


_PBLK = 2000


def _smax_body(xc_ref, b_ref, o_ref):
    i = pl.program_id(0)

    @pl.when(i == 0)
    def _():
        o_ref[...] = jnp.full_like(o_ref, -1e30)

    gid = lax.broadcasted_iota(jnp.int32, (1, N_GRAPHS), 1)
    oh = b_ref[...] == gid
    vals = jnp.where(oh, xc_ref[...], -1e30)
    o_ref[...] = jnp.maximum(o_ref[...], jnp.max(vals, axis=0, keepdims=True))


def _pool_body(xc_ref, b_ref, out_ref, smax_ref, den_ref, gx_ref):
    i = pl.program_id(0)

    @pl.when(i == 0)
    def _():
        den_ref[...] = jnp.zeros_like(den_ref)
        gx_ref[...] = jnp.zeros_like(gx_ref)

    gid = lax.broadcasted_iota(jnp.int32, (1, N_GRAPHS), 1)
    oh = b_ref[...] == gid
    smax_sel = jnp.max(jnp.where(oh, smax_ref[...], -1e30), axis=1, keepdims=True)
    ex = jnp.exp(xc_ref[...] - smax_sel)
    exoh = oh.astype(jnp.float32) * ex
    den_ref[...] += jnp.sum(exoh, axis=0, keepdims=True)
    gx_ref[...] += lax.dot_general(exoh, out_ref[...], (((0,), (0,)), ((), ())),
                                   preferred_element_type=jnp.float32)


def _pool(xc, batch_col, out):
    nb = N_EDGES // _PBLK
    colspec = pl.BlockSpec((_PBLK, 1), lambda i: (i, 0))
    smax = pl.pallas_call(
        _smax_body,
        grid=(nb,),
        in_specs=[colspec, colspec],
        out_specs=pl.BlockSpec((1, N_GRAPHS), lambda i: (0, 0)),
        out_shape=jax.ShapeDtypeStruct((1, N_GRAPHS), jnp.float32),
    )(xc, batch_col)
    den, gxr = pl.pallas_call(
        _pool_body,
        grid=(nb,),
        in_specs=[
            colspec,
            colspec,
            pl.BlockSpec((_PBLK, D), lambda i: (i, 0)),
            pl.BlockSpec((1, N_GRAPHS), lambda i: (0, 0)),
        ],
        out_specs=[
            pl.BlockSpec((1, N_GRAPHS), lambda i: (0, 0)),
            pl.BlockSpec((N_GRAPHS, D), lambda i: (0, 0)),
        ],
        out_shape=[
            jax.ShapeDtypeStruct((1, N_GRAPHS), jnp.float32),
            jax.ShapeDtypeStruct((N_GRAPHS, D), jnp.float32),
        ],
    )(xc, batch_col, out, smax)
    return gxr / jnp.maximum(den, 1e-30).T


def _stage(x, p, idx, use_prelu):
    i = str(idx)
    stats = _col_stats(x)
    pr = p['p' + i] if use_prelu else None
    return _bn_prelu_matmul(x, stats, p['g' + i], p['b' + i], p['W' + i], p['c' + i], pr)


def _linear_block_pallas(x, p):
    x1 = _stage(x, p, 1, False)
    x2 = _stage(x1, p, 2, True)
    x3 = _stage(x2, p, 3, True)
    xm = (x3 + x1) / 2.0
    x4 = _stage(xm, p, 4, True)
    xm2 = (x4 + xm) / 2.0
    return _stage(xm2, p, 5, True)


def kernel(x, edge_attr, params, edge_index, line_graph_edge_index, edge_index_batch):
    lg = line_graph_edge_index
    batch = edge_index_batch
    eu = x @ params['Wu']
    ev = x @ params['Wv']
    euv = edge_attr @ params['We']
    ea = (eu[edge_index[0]] + ev[edge_index[1]] + euv) / 3.0
    ssrc_pad, sdst_pad, tptr = _sort_lg(lg[1], lg[0], N_EDGES, _E)
    zeros_small = jnp.zeros((_zrows(_E // _NSUB), D), jnp.float32)
    batch_col = batch.reshape(-1, 1)
    out = ea
    out_list = []
    gout_list = []
    for _ in range(N_ITER):
        out = _sc_segsum(out, ea, ssrc_pad, sdst_pad, tptr, N_EDGES, _E)
        conv_agg = _sc_segsum(out, zeros_small, ssrc_pad, sdst_pad, tptr, N_EDGES, _E,
                              init_zero=True)
        xc = conv_agg @ params['Wrel'] + params['crel'] + out @ params['Wroot']
        gx = _pool(xc, batch_col, out)
        out_list.append(out)
        gout_list.append(jnp.tanh(gx @ params['Wgout'] + params['cgout']))
    gout_all = jnp.stack(gout_list, axis=-1)
    out_all = jnp.stack(out_list, axis=-1)
    sc = jnp.sum(gout_all * params['a'], axis=1, keepdims=True) + params['a_bias']
    sc = jax.nn.softmax(sc, axis=-1)
    sc_e = sc[batch]
    out = jnp.sum(out_all * sc_e, axis=-1)
    node_agg = jax.ops.segment_sum(out, edge_index[1], num_segments=N_NODES)
    h = x + node_agg
    return _linear_block_pallas(h, params)


# speedup vs baseline: 3.4880x; 1.1574x over previous
import dataclasses
import functools

import jax
import jax.numpy as jnp
from jax import lax
from jax.experimental import pallas as pl
from jax.experimental.pallas import tpu as pltpu
from jax.experimental.pallas import tpu_sc as plsc

N_NODES = 10000
N_EDGES = 320000
N_LG = 640000
N_GRAPHS = 128
D = 128
N_ITER = 3
SND = 6 * D

ROW_BLOCK = 1000


_E = 6400
_CHUNK = 128
_NSUB = 16


def _ptr_pad(n):
    return (n + 15) // 16 * 16


def _vext(vref, idx):
    base = pl.multiple_of((idx >> 4) << 4, 8)
    grp = vref[pl.ds(base, 16)]
    msk = lax.broadcasted_iota(jnp.int32, (16,), 0) == (idx & 15)
    return jnp.sum(jnp.where(msk, grp, 0), axis=0)


_IBLK = 1024
_NCH = _IBLK // _CHUNK


def _zrows(ER):
    for d in range(min(128, ER), 0, -1):
        if ER % d == 0:
            return d


def _seg_body(T, E, init_zero, src_hbm, base_hbm, ssrc_hbm, sdst_hbm, tptr_hbm,
              out_hbm, tptr_v, isrc_v, idst_v, ldstA, ldstB, rowsA, rowsB,
              zbuf, acc_sh, semA, semB):
    c = lax.axis_index("c")
    s = lax.axis_index("s")
    ER = E // _NSUB
    ZR = _zrows(ER)
    pltpu.sync_copy(tptr_hbm, tptr_v)
    if init_zero:
        pltpu.sync_copy(base_hbm, zbuf)
    n_my_tiles = (T - c + 1) // 2
    iota16 = lax.broadcasted_iota(jnp.int32, (16,), 0)

    def tile_body(i, carry):
        t = c + 2 * i
        tbase = t * E
        if init_zero:
            for k in range(ER // ZR):
                pltpu.sync_copy(zbuf, acc_sh.at[pl.ds(s * ER + k * ZR, ZR)])
        else:
            pltpu.sync_copy(base_hbm.at[pl.ds(tbase + s * ER, ER)],
                            acc_sh.at[pl.ds(s * ER, ER)])
        plsc.subcore_barrier()
        a = _vext(tptr_v, t)
        b = _vext(tptr_v, t + 1)
        lo = (a >> 3) << 3
        w8 = ((b + 7) >> 3) - (a >> 3)
        p0 = lo + ((w8 * s) >> 4) * 8
        p1 = lo + ((w8 * (s + 1)) >> 4) * 8
        nblk = (p1 - p0 + _IBLK - 1) // _IBLK

        rows = (rowsA, rowsB)
        sems = (semA, semB)
        lds = (ldstA, ldstB)

        def blk_body(bi, carry2):
            boff = pl.multiple_of(p0 + bi * _IBLK, 8)
            pltpu.sync_copy(ssrc_hbm.at[pl.ds(boff, _IBLK)], isrc_v)
            pltpu.sync_copy(sdst_hbm.at[pl.ds(boff, _IBLK)], idst_v)

            def compute_ldst(ci, lref):
                for v in range(_CHUNK // 16):
                    o = ci * _CHUNK + v * 16
                    dv = idst_v[pl.ds(o, 16)]
                    ld = dv - tbase
                    slot = boff + o + iota16
                    ok = (ld >= 0) & (ld < E) & (slot < p1)
                    lref[pl.ds(v * 16, 16)] = jnp.where(ok, ld, E)

            def finish(ci, cp):
                lref = lds[ci % 2]
                compute_ldst(ci, lref)
                cp.wait()
                pltpu.sync_copy(rows[ci % 2], acc_sh.at[lref], add=True)

            cps = [None, None]
            for ci in range(_NCH):
                cps[ci % 2] = pltpu.async_copy(
                    src_hbm.at[isrc_v.at[pl.ds(ci * _CHUNK, _CHUNK)]],
                    rows[ci % 2], sems[ci % 2])
                if ci >= 1:
                    finish(ci - 1, cps[(ci - 1) % 2])
            finish(_NCH - 1, cps[(_NCH - 1) % 2])
            return carry2

        lax.fori_loop(0, nblk, blk_body, 0)
        plsc.subcore_barrier()
        pltpu.sync_copy(acc_sh.at[pl.ds(s * ER, ER)],
                        out_hbm.at[pl.ds(tbase + s * ER, ER)])
        plsc.subcore_barrier()
        return carry

    lax.fori_loop(0, n_my_tiles, tile_body, 0)


def _sc_segsum(src, base, ssrc_pad, sdst_pad, tptr, n_out, E, init_zero=False):
    T = n_out // E
    mesh = plsc.VectorSubcoreMesh(core_axis_name="c", subcore_axis_name="s")
    body = functools.partial(_seg_body, T, E, init_zero)
    cp = pltpu.CompilerParams()
    if "needs_layout_passes" in pltpu.CompilerParams.__dataclass_fields__:
        cp = dataclasses.replace(cp, needs_layout_passes=False)
    f = pl.kernel(
        body,
        out_type=jax.ShapeDtypeStruct((n_out, D), jnp.float32),
        mesh=mesh,
        compiler_params=cp,
        scratch_types=[
            pltpu.VMEM((_ptr_pad(T + 1),), jnp.int32),
            pltpu.VMEM((_IBLK,), jnp.int32),
            pltpu.VMEM((_IBLK,), jnp.int32),
            pltpu.VMEM((_CHUNK,), jnp.int32),
            pltpu.VMEM((_CHUNK,), jnp.int32),
            pltpu.VMEM((_CHUNK, D), jnp.float32),
            pltpu.VMEM((_CHUNK, D), jnp.float32),
            pltpu.VMEM((_zrows(E // _NSUB), D), jnp.float32),
            pltpu.VMEM_SHARED((E + 8, D), jnp.float32),
            pltpu.SemaphoreType.DMA,
            pltpu.SemaphoreType.DMA,
        ],
    )
    return f(src, base, ssrc_pad, sdst_pad, tptr)


def _sort_lg(lg_dst, lg_src, n_out, E):
    sdst, ssrc = lax.sort((lg_dst, lg_src), dimension=0, num_keys=1)
    T = n_out // E
    tptr = jnp.searchsorted(sdst, jnp.arange(T + 1, dtype=jnp.int32) * E).astype(jnp.int32)
    tptr = jnp.concatenate([tptr, jnp.full((_ptr_pad(T + 1) - (T + 1),), sdst.shape[0], jnp.int32)])
    ssrc_pad = jnp.concatenate([ssrc, jnp.zeros((2 * _IBLK,), jnp.int32)])
    sdst_pad = jnp.concatenate([sdst, jnp.full((2 * _IBLK,), n_out, jnp.int32)])
    return ssrc_pad, sdst_pad, tptr


def _stats_body(x_ref, o_ref):
    i = pl.program_id(0)

    @pl.when(i == 0)
    def _():
        o_ref[...] = jnp.zeros_like(o_ref)

    xb = x_ref[...]
    o_ref[0, :] += jnp.sum(xb, axis=0)
    o_ref[1, :] += jnp.sum(xb * xb, axis=0)


def _col_stats(x):
    n, c = x.shape
    return pl.pallas_call(
        _stats_body,
        grid=(n // ROW_BLOCK,),
        in_specs=[pl.BlockSpec((ROW_BLOCK, c), lambda i: (i, 0))],
        out_specs=pl.BlockSpec((2, c), lambda i: (0, 0)),
        out_shape=jax.ShapeDtypeStruct((2, c), jnp.float32),
    )(x)


def _apply_body(x_ref, s_ref, g_ref, b_ref, w_ref, c_ref, p_ref, o_ref, *, n, use_prelu):
    m = s_ref[0, :] / n
    v = s_ref[1, :] / n - m * m
    xn = (x_ref[...] - m[None, :]) * (g_ref[0, :] / jnp.sqrt(v + 1e-5))[None, :] + b_ref[0, :][None, :]
    if use_prelu:
        p = p_ref[0, 0]
        xn = jnp.where(xn >= 0, xn, p * xn)
    o_ref[...] = jnp.dot(xn, w_ref[...], preferred_element_type=jnp.float32) + c_ref[0, :][None, :]


def _bn_prelu_matmul(x, stats, g, b, w, c, p):
    n, cin = x.shape
    cout = w.shape[1]
    use_prelu = p is not None
    if p is None:
        p_arr = jnp.zeros((1, 1), jnp.float32)
    else:
        p_arr = jnp.asarray(p, jnp.float32).reshape(1, 1)
    body = functools.partial(_apply_body, n=float(n), use_prelu=use_prelu)
    return pl.pallas_call(
        body,
        grid=(n // ROW_BLOCK,),
        in_specs=[
            pl.BlockSpec((ROW_BLOCK, cin), lambda i: (i, 0)),
            pl.BlockSpec((2, cin), lambda i: (0, 0)),
            pl.BlockSpec((1, cin), lambda i: (0, 0)),
            pl.BlockSpec((1, cin), lambda i: (0, 0)),
            pl.BlockSpec((cin, cout), lambda i: (0, 0)),
            pl.BlockSpec((1, cout), lambda i: (0, 0)),
            pl.BlockSpec((1, 1), lambda i: (0, 0)),
        ],
        out_specs=pl.BlockSpec((ROW_BLOCK, cout), lambda i: (i, 0)),
        out_shape=jax.ShapeDtypeStruct((n, cout), jnp.float32),
    )(x, stats, g.reshape(1, -1), b.reshape(1, -1), w, c.reshape(1, -1), p_arr)


_PBLK = 2000


def _smax_body(xc_ref, b_ref, o_ref):
    i = pl.program_id(0)

    @pl.when(i == 0)
    def _():
        o_ref[...] = jnp.full_like(o_ref, -1e30)

    gid = lax.broadcasted_iota(jnp.int32, (1, N_GRAPHS), 1)
    oh = b_ref[...] == gid
    vals = jnp.where(oh, xc_ref[...], -1e30)
    o_ref[...] = jnp.maximum(o_ref[...], jnp.max(vals, axis=0, keepdims=True))


def _pool_body(xc_ref, b_ref, out_ref, smax_ref, den_ref, gx_ref):
    i = pl.program_id(0)

    @pl.when(i == 0)
    def _():
        den_ref[...] = jnp.zeros_like(den_ref)
        gx_ref[...] = jnp.zeros_like(gx_ref)

    gid = lax.broadcasted_iota(jnp.int32, (1, N_GRAPHS), 1)
    oh = b_ref[...] == gid
    smax_sel = jnp.max(jnp.where(oh, smax_ref[...], -1e30), axis=1, keepdims=True)
    ex = jnp.exp(xc_ref[...] - smax_sel)
    exoh = oh.astype(jnp.float32) * ex
    den_ref[...] += jnp.sum(exoh, axis=0, keepdims=True)
    gx_ref[...] += lax.dot_general(exoh, out_ref[...], (((0,), (0,)), ((), ())),
                                   preferred_element_type=jnp.float32)


def _pool(xc, batch_col, out):
    nb = N_EDGES // _PBLK
    colspec = pl.BlockSpec((_PBLK, 1), lambda i: (i, 0))
    smax = pl.pallas_call(
        _smax_body,
        grid=(nb,),
        in_specs=[colspec, colspec],
        out_specs=pl.BlockSpec((1, N_GRAPHS), lambda i: (0, 0)),
        out_shape=jax.ShapeDtypeStruct((1, N_GRAPHS), jnp.float32),
    )(xc, batch_col)
    den, gxr = pl.pallas_call(
        _pool_body,
        grid=(nb,),
        in_specs=[
            colspec,
            colspec,
            pl.BlockSpec((_PBLK, D), lambda i: (i, 0)),
            pl.BlockSpec((1, N_GRAPHS), lambda i: (0, 0)),
        ],
        out_specs=[
            pl.BlockSpec((1, N_GRAPHS), lambda i: (0, 0)),
            pl.BlockSpec((N_GRAPHS, D), lambda i: (0, 0)),
        ],
        out_shape=[
            jax.ShapeDtypeStruct((1, N_GRAPHS), jnp.float32),
            jax.ShapeDtypeStruct((N_GRAPHS, D), jnp.float32),
        ],
    )(xc, batch_col, out, smax)
    return gxr / jnp.maximum(den, 1e-30).T


def _comb_body(o1_ref, o2_ref, o3_ref, b_ref, w1_ref, w2_ref, w3_ref, out_ref):
    gid = lax.broadcasted_iota(jnp.int32, (1, N_GRAPHS), 1)
    oh = (b_ref[...] == gid).astype(jnp.float32)
    dn = (((1,), (0,)), ((), ()))
    w1 = lax.dot_general(oh, w1_ref[...], dn, preferred_element_type=jnp.float32)
    w2 = lax.dot_general(oh, w2_ref[...], dn, preferred_element_type=jnp.float32)
    w3 = lax.dot_general(oh, w3_ref[...], dn, preferred_element_type=jnp.float32)
    out_ref[...] = o1_ref[...] * w1 + o2_ref[...] * w2 + o3_ref[...] * w3


def _combine(outs, batch_col, scg):
    nb = N_EDGES // _PBLK
    rowspec = pl.BlockSpec((_PBLK, D), lambda i: (i, 0))
    colspec = pl.BlockSpec((_PBLK, 1), lambda i: (i, 0))
    wspec = pl.BlockSpec((N_GRAPHS, 1), lambda i: (0, 0))
    return pl.pallas_call(
        _comb_body,
        grid=(nb,),
        in_specs=[rowspec, rowspec, rowspec, colspec, wspec, wspec, wspec],
        out_specs=rowspec,
        out_shape=jax.ShapeDtypeStruct((N_EDGES, D), jnp.float32),
    )(outs[0], outs[1], outs[2], batch_col,
      scg[:, 0:1], scg[:, 1:2], scg[:, 2:3])


_NODE_PAD = 10240


def _nodeagg_body(src_hbm, ei1_hbm, zeros_hbm, out_hbm,
                  idxA, idxB, rowsA, rowsB, zbuf, acc_sh, semA, semB):
    c = lax.axis_index("c")
    s = lax.axis_index("s")
    ER = _NODE_PAD // _NSUB
    ZR = 64
    idx = (idxA, idxB)
    rows = (rowsA, rowsB)
    sems = (semA, semB)
    pltpu.sync_copy(zeros_hbm, zbuf)
    for k in range(ER // ZR):
        pltpu.sync_copy(zbuf, acc_sh.at[pl.ds(s * ER + k * ZR, ZR)])
    plsc.subcore_barrier()

    def chunk_io(j, buf):
        ci = c + 2 * (s + _NSUB * j)
        off = pl.multiple_of(ci * _CHUNK, 8)
        pltpu.sync_copy(ei1_hbm.at[pl.ds(off, _CHUNK)], idx[buf])
        return pltpu.async_copy(src_hbm.at[pl.ds(off, _CHUNK)], rows[buf], sems[buf])

    def finishc(cp, buf):
        cp.wait()
        pltpu.sync_copy(rows[buf], acc_sh.at[idx[buf]], add=True)

    n_pairs = (N_EDGES // _CHUNK) // (2 * _NSUB * 2)

    def pair(k, carry):
        cp0 = chunk_io(2 * k, 0)
        cp1 = chunk_io(2 * k + 1, 1)
        finishc(cp0, 0)
        finishc(cp1, 1)
        return carry

    lax.fori_loop(0, n_pairs, pair, 0)

    @pl.when(s < 2)
    def _():
        finishc(chunk_io(2 * n_pairs, 0), 0)

    plsc.subcore_barrier()
    pltpu.sync_copy(acc_sh.at[pl.ds(s * ER, ER)],
                    out_hbm.at[c].at[pl.ds(s * ER, ER)])


def _sc_nodeagg(src, ei1):
    mesh = plsc.VectorSubcoreMesh(core_axis_name="c", subcore_axis_name="s")
    cp = pltpu.CompilerParams()
    if "needs_layout_passes" in pltpu.CompilerParams.__dataclass_fields__:
        cp = dataclasses.replace(cp, needs_layout_passes=False)
    f = pl.kernel(
        _nodeagg_body,
        out_type=jax.ShapeDtypeStruct((2, _NODE_PAD, D), jnp.float32),
        mesh=mesh,
        compiler_params=cp,
        scratch_types=[
            pltpu.VMEM((_CHUNK,), jnp.int32),
            pltpu.VMEM((_CHUNK,), jnp.int32),
            pltpu.VMEM((_CHUNK, D), jnp.float32),
            pltpu.VMEM((_CHUNK, D), jnp.float32),
            pltpu.VMEM((64, D), jnp.float32),
            pltpu.VMEM_SHARED((_NODE_PAD, D), jnp.float32),
            pltpu.SemaphoreType.DMA,
            pltpu.SemaphoreType.DMA,
        ],
    )
    parts = f(src, ei1, jnp.zeros((64, D), jnp.float32))
    return parts[0, :N_NODES] + parts[1, :N_NODES]


def _stage(x, p, idx, use_prelu):
    i = str(idx)
    stats = _col_stats(x)
    pr = p['p' + i] if use_prelu else None
    return _bn_prelu_matmul(x, stats, p['g' + i], p['b' + i], p['W' + i], p['c' + i], pr)


def _linear_block_pallas(x, p):
    x1 = _stage(x, p, 1, False)
    x2 = _stage(x1, p, 2, True)
    x3 = _stage(x2, p, 3, True)
    xm = (x3 + x1) / 2.0
    x4 = _stage(xm, p, 4, True)
    xm2 = (x4 + xm) / 2.0
    return _stage(xm2, p, 5, True)


def kernel(x, edge_attr, params, edge_index, line_graph_edge_index, edge_index_batch):
    lg = line_graph_edge_index
    batch = edge_index_batch
    eu = x @ params['Wu']
    ev = x @ params['Wv']
    euv = edge_attr @ params['We']
    ea = (eu[edge_index[0]] + ev[edge_index[1]] + euv) / 3.0
    ssrc_pad, sdst_pad, tptr = _sort_lg(lg[1], lg[0], N_EDGES, _E)
    zeros_small = jnp.zeros((_zrows(_E // _NSUB), D), jnp.float32)
    batch_col = batch.reshape(-1, 1)
    out = ea
    out_list = []
    gout_list = []
    for _ in range(N_ITER):
        out = _sc_segsum(out, ea, ssrc_pad, sdst_pad, tptr, N_EDGES, _E)
        conv_agg = _sc_segsum(out, zeros_small, ssrc_pad, sdst_pad, tptr, N_EDGES, _E,
                              init_zero=True)
        xc = conv_agg @ params['Wrel'] + params['crel'] + out @ params['Wroot']
        gx = _pool(xc, batch_col, out)
        out_list.append(out)
        gout_list.append(jnp.tanh(gx @ params['Wgout'] + params['cgout']))
    gout_all = jnp.stack(gout_list, axis=-1)
    sc = jnp.sum(gout_all * params['a'], axis=1, keepdims=True) + params['a_bias']
    sc = jax.nn.softmax(sc, axis=-1)
    scg = sc[:, 0, :]
    out = _combine(out_list, batch_col, scg)
    node_agg = _sc_nodeagg(out, edge_index[1])
    h = x + node_agg
    return _linear_block_pallas(h, params)


# speedup vs baseline: 4.1879x; 1.2007x over previous
import dataclasses
import functools

import jax
import jax.numpy as jnp
from jax import lax
from jax.experimental import pallas as pl
from jax.experimental.pallas import tpu as pltpu
from jax.experimental.pallas import tpu_sc as plsc

N_NODES = 10000
N_EDGES = 320000
N_LG = 640000
N_GRAPHS = 128
D = 128
N_ITER = 3
SND = 6 * D

ROW_BLOCK = 1000


_E = 6400
_CHUNK = 128
_NSUB = 16


def _ptr_pad(n):
    return (n + 15) // 16 * 16


def _vext(vref, idx):
    base = pl.multiple_of((idx >> 4) << 4, 8)
    grp = vref[pl.ds(base, 16)]
    msk = lax.broadcasted_iota(jnp.int32, (16,), 0) == (idx & 15)
    return jnp.sum(jnp.where(msk, grp, 0), axis=0)


_IBLK = 1024
_NCH = _IBLK // _CHUNK


def _zrows(ER):
    for d in range(min(128, ER), 0, -1):
        if ER % d == 0:
            return d


def _seg_body(T, E, init_zero, src_hbm, base_hbm, ssrc_hbm, sdst_hbm, tptr_hbm,
              out_hbm, tptr_v, isrc_v, idst_v, ldstA, ldstB, rowsA, rowsB,
              zbuf, acc_sh, semA, semB):
    c = lax.axis_index("c")
    s = lax.axis_index("s")
    ER = E // _NSUB
    ZR = _zrows(ER)
    pltpu.sync_copy(tptr_hbm, tptr_v)
    if init_zero:
        pltpu.sync_copy(base_hbm, zbuf)
    n_my_tiles = (T - c + 1) // 2
    iota16 = lax.broadcasted_iota(jnp.int32, (16,), 0)

    def tile_body(i, carry):
        t = c + 2 * i
        tbase = t * E
        if init_zero:
            for k in range(ER // ZR):
                pltpu.sync_copy(zbuf, acc_sh.at[pl.ds(s * ER + k * ZR, ZR)])
        else:
            pltpu.sync_copy(base_hbm.at[pl.ds(tbase + s * ER, ER)],
                            acc_sh.at[pl.ds(s * ER, ER)])
        plsc.subcore_barrier()
        a = _vext(tptr_v, t)
        b = _vext(tptr_v, t + 1)
        lo = (a >> 3) << 3
        w8 = ((b + 7) >> 3) - (a >> 3)
        p0 = lo + ((w8 * s) >> 4) * 8
        p1 = lo + ((w8 * (s + 1)) >> 4) * 8
        nblk = (p1 - p0 + _IBLK - 1) // _IBLK

        rows = (rowsA, rowsB)
        sems = (semA, semB)
        lds = (ldstA, ldstB)

        def blk_body(bi, carry2):
            boff = pl.multiple_of(p0 + bi * _IBLK, 8)
            pltpu.sync_copy(ssrc_hbm.at[pl.ds(boff, _IBLK)], isrc_v)
            pltpu.sync_copy(sdst_hbm.at[pl.ds(boff, _IBLK)], idst_v)

            def compute_ldst(ci, lref):
                for v in range(_CHUNK // 16):
                    o = ci * _CHUNK + v * 16
                    dv = idst_v[pl.ds(o, 16)]
                    ld = dv - tbase
                    slot = boff + o + iota16
                    ok = (ld >= 0) & (ld < E) & (slot < p1)
                    lref[pl.ds(v * 16, 16)] = jnp.where(ok, ld, E)

            def finish(ci, cp):
                lref = lds[ci % 2]
                compute_ldst(ci, lref)
                cp.wait()
                pltpu.sync_copy(rows[ci % 2], acc_sh.at[lref], add=True)

            cps = [None, None]
            for ci in range(_NCH):
                cps[ci % 2] = pltpu.async_copy(
                    src_hbm.at[isrc_v.at[pl.ds(ci * _CHUNK, _CHUNK)]],
                    rows[ci % 2], sems[ci % 2])
                if ci >= 1:
                    finish(ci - 1, cps[(ci - 1) % 2])
            finish(_NCH - 1, cps[(_NCH - 1) % 2])
            return carry2

        lax.fori_loop(0, nblk, blk_body, 0)
        plsc.subcore_barrier()
        pltpu.sync_copy(acc_sh.at[pl.ds(s * ER, ER)],
                        out_hbm.at[pl.ds(tbase + s * ER, ER)])
        plsc.subcore_barrier()
        return carry

    lax.fori_loop(0, n_my_tiles, tile_body, 0)


def _sc_segsum(src, base, ssrc_pad, sdst_pad, tptr, n_out, E, init_zero=False):
    T = n_out // E
    mesh = plsc.VectorSubcoreMesh(core_axis_name="c", subcore_axis_name="s")
    body = functools.partial(_seg_body, T, E, init_zero)
    cp = pltpu.CompilerParams()
    if "needs_layout_passes" in pltpu.CompilerParams.__dataclass_fields__:
        cp = dataclasses.replace(cp, needs_layout_passes=False)
    f = pl.kernel(
        body,
        out_type=jax.ShapeDtypeStruct((n_out, D), jnp.float32),
        mesh=mesh,
        compiler_params=cp,
        scratch_types=[
            pltpu.VMEM((_ptr_pad(T + 1),), jnp.int32),
            pltpu.VMEM((_IBLK,), jnp.int32),
            pltpu.VMEM((_IBLK,), jnp.int32),
            pltpu.VMEM((_CHUNK,), jnp.int32),
            pltpu.VMEM((_CHUNK,), jnp.int32),
            pltpu.VMEM((_CHUNK, D), jnp.float32),
            pltpu.VMEM((_CHUNK, D), jnp.float32),
            pltpu.VMEM((_zrows(E // _NSUB), D), jnp.float32),
            pltpu.VMEM_SHARED((E + 8, D), jnp.float32),
            pltpu.SemaphoreType.DMA,
            pltpu.SemaphoreType.DMA,
        ],
    )
    return f(src, base, ssrc_pad, sdst_pad, tptr)


def _sort_lg(lg_dst, lg_src, n_out, E):
    sdst, ssrc = lax.sort((lg_dst, lg_src), dimension=0, num_keys=1)
    T = n_out // E
    tptr = jnp.searchsorted(sdst, jnp.arange(T + 1, dtype=jnp.int32) * E).astype(jnp.int32)
    tptr = jnp.concatenate([tptr, jnp.full((_ptr_pad(T + 1) - (T + 1),), sdst.shape[0], jnp.int32)])
    ssrc_pad = jnp.concatenate([ssrc, jnp.zeros((2 * _IBLK,), jnp.int32)])
    sdst_pad = jnp.concatenate([sdst, jnp.full((2 * _IBLK,), n_out, jnp.int32)])
    return ssrc_pad, sdst_pad, tptr


def _stats_body(x_ref, o_ref):
    i = pl.program_id(0)

    @pl.when(i == 0)
    def _():
        o_ref[...] = jnp.zeros_like(o_ref)

    xb = x_ref[...]
    o_ref[0, :] += jnp.sum(xb, axis=0)
    o_ref[1, :] += jnp.sum(xb * xb, axis=0)


def _col_stats(x):
    n, c = x.shape
    return pl.pallas_call(
        _stats_body,
        grid=(n // ROW_BLOCK,),
        in_specs=[pl.BlockSpec((ROW_BLOCK, c), lambda i: (i, 0))],
        out_specs=pl.BlockSpec((2, c), lambda i: (0, 0)),
        out_shape=jax.ShapeDtypeStruct((2, c), jnp.float32),
    )(x)


def _apply_body(x_ref, s_ref, g_ref, b_ref, w_ref, c_ref, p_ref, o_ref, *, n, use_prelu):
    m = s_ref[0, :] / n
    v = s_ref[1, :] / n - m * m
    xn = (x_ref[...] - m[None, :]) * (g_ref[0, :] / jnp.sqrt(v + 1e-5))[None, :] + b_ref[0, :][None, :]
    if use_prelu:
        p = p_ref[0, 0]
        xn = jnp.where(xn >= 0, xn, p * xn)
    o_ref[...] = jnp.dot(xn, w_ref[...], preferred_element_type=jnp.float32) + c_ref[0, :][None, :]


def _bn_prelu_matmul(x, stats, g, b, w, c, p):
    n, cin = x.shape
    cout = w.shape[1]
    use_prelu = p is not None
    if p is None:
        p_arr = jnp.zeros((1, 1), jnp.float32)
    else:
        p_arr = jnp.asarray(p, jnp.float32).reshape(1, 1)
    body = functools.partial(_apply_body, n=float(n), use_prelu=use_prelu)
    return pl.pallas_call(
        body,
        grid=(n // ROW_BLOCK,),
        in_specs=[
            pl.BlockSpec((ROW_BLOCK, cin), lambda i: (i, 0)),
            pl.BlockSpec((2, cin), lambda i: (0, 0)),
            pl.BlockSpec((1, cin), lambda i: (0, 0)),
            pl.BlockSpec((1, cin), lambda i: (0, 0)),
            pl.BlockSpec((cin, cout), lambda i: (0, 0)),
            pl.BlockSpec((1, cout), lambda i: (0, 0)),
            pl.BlockSpec((1, 1), lambda i: (0, 0)),
        ],
        out_specs=pl.BlockSpec((ROW_BLOCK, cout), lambda i: (i, 0)),
        out_shape=jax.ShapeDtypeStruct((n, cout), jnp.float32),
    )(x, stats, g.reshape(1, -1), b.reshape(1, -1), w, c.reshape(1, -1), p_arr)


_PBLK = 2000


def _smax_body(xc_ref, b_ref, o_ref):
    i = pl.program_id(0)

    @pl.when(i == 0)
    def _():
        o_ref[...] = jnp.full_like(o_ref, -1e30)

    gid = lax.broadcasted_iota(jnp.int32, (1, N_GRAPHS), 1)
    oh = b_ref[...] == gid
    vals = jnp.where(oh, xc_ref[...], -1e30)
    o_ref[...] = jnp.maximum(o_ref[...], jnp.max(vals, axis=0, keepdims=True))


def _pool_body(xc_ref, b_ref, out_ref, smax_ref, den_ref, gx_ref):
    i = pl.program_id(0)

    @pl.when(i == 0)
    def _():
        den_ref[...] = jnp.zeros_like(den_ref)
        gx_ref[...] = jnp.zeros_like(gx_ref)

    gid = lax.broadcasted_iota(jnp.int32, (1, N_GRAPHS), 1)
    oh = b_ref[...] == gid
    smax_sel = jnp.max(jnp.where(oh, smax_ref[...], -1e30), axis=1, keepdims=True)
    ex = jnp.exp(xc_ref[...] - smax_sel)
    exoh = oh.astype(jnp.float32) * ex
    den_ref[...] += jnp.sum(exoh, axis=0, keepdims=True)
    gx_ref[...] += lax.dot_general(exoh, out_ref[...], (((0,), (0,)), ((), ())),
                                   preferred_element_type=jnp.float32)


def _pool(xc, batch_col, out):
    nb = N_EDGES // _PBLK
    colspec = pl.BlockSpec((_PBLK, 1), lambda i: (i, 0))
    smax = pl.pallas_call(
        _smax_body,
        grid=(nb,),
        in_specs=[colspec, colspec],
        out_specs=pl.BlockSpec((1, N_GRAPHS), lambda i: (0, 0)),
        out_shape=jax.ShapeDtypeStruct((1, N_GRAPHS), jnp.float32),
    )(xc, batch_col)
    den, gxr = pl.pallas_call(
        _pool_body,
        grid=(nb,),
        in_specs=[
            colspec,
            colspec,
            pl.BlockSpec((_PBLK, D), lambda i: (i, 0)),
            pl.BlockSpec((1, N_GRAPHS), lambda i: (0, 0)),
        ],
        out_specs=[
            pl.BlockSpec((1, N_GRAPHS), lambda i: (0, 0)),
            pl.BlockSpec((N_GRAPHS, D), lambda i: (0, 0)),
        ],
        out_shape=[
            jax.ShapeDtypeStruct((1, N_GRAPHS), jnp.float32),
            jax.ShapeDtypeStruct((N_GRAPHS, D), jnp.float32),
        ],
    )(xc, batch_col, out, smax)
    return gxr / jnp.maximum(den, 1e-30).T


def _comb_body(o1_ref, o2_ref, o3_ref, b_ref, w1_ref, w2_ref, w3_ref, out_ref):
    gid = lax.broadcasted_iota(jnp.int32, (1, N_GRAPHS), 1)
    oh = (b_ref[...] == gid).astype(jnp.float32)
    dn = (((1,), (0,)), ((), ()))
    w1 = lax.dot_general(oh, w1_ref[...], dn, preferred_element_type=jnp.float32)
    w2 = lax.dot_general(oh, w2_ref[...], dn, preferred_element_type=jnp.float32)
    w3 = lax.dot_general(oh, w3_ref[...], dn, preferred_element_type=jnp.float32)
    out_ref[...] = o1_ref[...] * w1 + o2_ref[...] * w2 + o3_ref[...] * w3


def _combine(outs, batch_col, scg):
    nb = N_EDGES // _PBLK
    rowspec = pl.BlockSpec((_PBLK, D), lambda i: (i, 0))
    colspec = pl.BlockSpec((_PBLK, 1), lambda i: (i, 0))
    wspec = pl.BlockSpec((N_GRAPHS, 1), lambda i: (0, 0))
    return pl.pallas_call(
        _comb_body,
        grid=(nb,),
        in_specs=[rowspec, rowspec, rowspec, colspec, wspec, wspec, wspec],
        out_specs=rowspec,
        out_shape=jax.ShapeDtypeStruct((N_EDGES, D), jnp.float32),
    )(outs[0], outs[1], outs[2], batch_col,
      scg[:, 0:1], scg[:, 1:2], scg[:, 2:3])


_NODE_PAD = 10240


def _nodeagg_body(src_hbm, ei1_hbm, zeros_hbm, out_hbm,
                  idxA, idxB, rowsA, rowsB, zbuf, acc_sh, semA, semB):
    c = lax.axis_index("c")
    s = lax.axis_index("s")
    ER = _NODE_PAD // _NSUB
    ZR = 64
    idx = (idxA, idxB)
    rows = (rowsA, rowsB)
    sems = (semA, semB)
    pltpu.sync_copy(zeros_hbm, zbuf)
    for k in range(ER // ZR):
        pltpu.sync_copy(zbuf, acc_sh.at[pl.ds(s * ER + k * ZR, ZR)])
    plsc.subcore_barrier()

    def chunk_io(j, buf):
        ci = c + 2 * (s + _NSUB * j)
        off = pl.multiple_of(ci * _CHUNK, 8)
        pltpu.sync_copy(ei1_hbm.at[pl.ds(off, _CHUNK)], idx[buf])
        return pltpu.async_copy(src_hbm.at[pl.ds(off, _CHUNK)], rows[buf], sems[buf])

    def finishc(cp, buf):
        cp.wait()
        pltpu.sync_copy(rows[buf], acc_sh.at[idx[buf]], add=True)

    n_pairs = (N_EDGES // _CHUNK) // (2 * _NSUB * 2)

    def pair(k, carry):
        cp0 = chunk_io(2 * k, 0)
        cp1 = chunk_io(2 * k + 1, 1)
        finishc(cp0, 0)
        finishc(cp1, 1)
        return carry

    lax.fori_loop(0, n_pairs, pair, 0)

    @pl.when(s < 2)
    def _():
        finishc(chunk_io(2 * n_pairs, 0), 0)

    plsc.subcore_barrier()
    pltpu.sync_copy(acc_sh.at[pl.ds(s * ER, ER)],
                    out_hbm.at[c].at[pl.ds(s * ER, ER)])


def _sc_nodeagg(src, ei1):
    mesh = plsc.VectorSubcoreMesh(core_axis_name="c", subcore_axis_name="s")
    cp = pltpu.CompilerParams()
    if "needs_layout_passes" in pltpu.CompilerParams.__dataclass_fields__:
        cp = dataclasses.replace(cp, needs_layout_passes=False)
    f = pl.kernel(
        _nodeagg_body,
        out_type=jax.ShapeDtypeStruct((2, _NODE_PAD, D), jnp.float32),
        mesh=mesh,
        compiler_params=cp,
        scratch_types=[
            pltpu.VMEM((_CHUNK,), jnp.int32),
            pltpu.VMEM((_CHUNK,), jnp.int32),
            pltpu.VMEM((_CHUNK, D), jnp.float32),
            pltpu.VMEM((_CHUNK, D), jnp.float32),
            pltpu.VMEM((64, D), jnp.float32),
            pltpu.VMEM_SHARED((_NODE_PAD, D), jnp.float32),
            pltpu.SemaphoreType.DMA,
            pltpu.SemaphoreType.DMA,
        ],
    )
    parts = f(src, ei1, jnp.zeros((64, D), jnp.float32))
    return parts[0, :N_NODES] + parts[1, :N_NODES]


def _ea_body(eu_hbm, ev_hbm, euv_hbm, ei0_hbm, ei1_hbm, out_hbm,
             i0A, i0B, i1A, i1B, rA0, rA1, rB0, rB1, rC0, rC1,
             sA0, sA1, sB0, sB1, sC0, sC1, sW0, sW1):
    c = lax.axis_index("c")
    s = lax.axis_index("s")
    i0 = (i0A, i0B)
    i1 = (i1A, i1B)
    rA = (rA0, rA1)
    rB = (rB0, rB1)
    rC = (rC0, rC1)
    sA = (sA0, sA1)
    sB = (sB0, sB1)
    sC = (sC0, sC1)
    sW = (sW0, sW1)

    def chunk_io(j, h):
        ci = c + 2 * (s + _NSUB * j)
        off = pl.multiple_of(ci * _CHUNK, 8)
        pltpu.sync_copy(ei0_hbm.at[pl.ds(off, _CHUNK)], i0[h])
        pltpu.sync_copy(ei1_hbm.at[pl.ds(off, _CHUNK)], i1[h])
        cpa = pltpu.async_copy(eu_hbm.at[i0[h]], rA[h], sA[h])
        cpb = pltpu.async_copy(ev_hbm.at[i1[h]], rB[h], sB[h])
        cpc = pltpu.async_copy(euv_hbm.at[pl.ds(off, _CHUNK)], rC[h], sC[h])
        return (cpa, cpb, cpc, off)

    def finishc(cps, h):
        cpa, cpb, cpc, off = cps
        cpa.wait()
        cpb.wait()
        cpc.wait()

        def row(r, carry):
            for g in range(D // 16):
                sl = pl.ds(g * 16, 16)
                rC[h][r, sl] = (rA[h][r, sl] + rB[h][r, sl] + rC[h][r, sl]) * (1.0 / 3.0)
            return carry

        lax.fori_loop(0, _CHUNK, row, 0)
        return pltpu.async_copy(rC[h], out_hbm.at[pl.ds(off, _CHUNK)], sW[h])

    n_pairs = (N_EDGES // _CHUNK) // (2 * _NSUB * 2)
    wr = [None, None]

    def pair(k, carry):
        cps0 = chunk_io(2 * k, 0)
        cps1 = chunk_io(2 * k + 1, 1)
        w0 = finishc(cps0, 0)
        w1 = finishc(cps1, 1)
        w0.wait()
        w1.wait()
        return carry

    lax.fori_loop(0, n_pairs, pair, 0)

    @pl.when(s < 2)
    def _():
        finishc(chunk_io(2 * n_pairs, 0), 0).wait()


def _sc_ea(eu, ev, euv, ei0, ei1):
    mesh = plsc.VectorSubcoreMesh(core_axis_name="c", subcore_axis_name="s")
    cp = pltpu.CompilerParams()
    if "needs_layout_passes" in pltpu.CompilerParams.__dataclass_fields__:
        cp = dataclasses.replace(cp, needs_layout_passes=False)
    f = pl.kernel(
        _ea_body,
        out_type=jax.ShapeDtypeStruct((N_EDGES, D), jnp.float32),
        mesh=mesh,
        compiler_params=cp,
        scratch_types=(
            [pltpu.VMEM((_CHUNK,), jnp.int32)] * 4
            + [pltpu.VMEM((_CHUNK, D), jnp.float32)] * 6
            + [pltpu.SemaphoreType.DMA] * 8
        ),
    )
    return f(eu, ev, euv, ei0, ei1)


def _stage(x, p, idx, use_prelu):
    i = str(idx)
    stats = _col_stats(x)
    pr = p['p' + i] if use_prelu else None
    return _bn_prelu_matmul(x, stats, p['g' + i], p['b' + i], p['W' + i], p['c' + i], pr)


def _linear_block_pallas(x, p):
    x1 = _stage(x, p, 1, False)
    x2 = _stage(x1, p, 2, True)
    x3 = _stage(x2, p, 3, True)
    xm = (x3 + x1) / 2.0
    x4 = _stage(xm, p, 4, True)
    xm2 = (x4 + xm) / 2.0
    return _stage(xm2, p, 5, True)


def kernel(x, edge_attr, params, edge_index, line_graph_edge_index, edge_index_batch):
    lg = line_graph_edge_index
    batch = edge_index_batch
    eu = x @ params['Wu']
    ev = x @ params['Wv']
    euv = edge_attr @ params['We']
    ea = _sc_ea(eu, ev, euv, edge_index[0], edge_index[1])
    ssrc_pad, sdst_pad, tptr = _sort_lg(lg[1], lg[0], N_EDGES, _E)
    zeros_small = jnp.zeros((_zrows(_E // _NSUB), D), jnp.float32)
    batch_col = batch.reshape(-1, 1)
    out = ea
    out_list = []
    gout_list = []
    for _ in range(N_ITER):
        out = _sc_segsum(out, ea, ssrc_pad, sdst_pad, tptr, N_EDGES, _E)
        conv_agg = _sc_segsum(out, zeros_small, ssrc_pad, sdst_pad, tptr, N_EDGES, _E,
                              init_zero=True)
        xc = conv_agg @ params['Wrel'] + params['crel'] + out @ params['Wroot']
        gx = _pool(xc, batch_col, out)
        out_list.append(out)
        gout_list.append(jnp.tanh(gx @ params['Wgout'] + params['cgout']))
    gout_all = jnp.stack(gout_list, axis=-1)
    sc = jnp.sum(gout_all * params['a'], axis=1, keepdims=True) + params['a_bias']
    sc = jax.nn.softmax(sc, axis=-1)
    scg = sc[:, 0, :]
    out = _combine(out_list, batch_col, scg)
    node_agg = _sc_nodeagg(out, edge_index[1])
    h = x + node_agg
    return _linear_block_pallas(h, params)


# speedup vs baseline: 4.4582x; 1.0645x over previous
import dataclasses
import functools

import jax
import jax.numpy as jnp
from jax import lax
from jax.experimental import pallas as pl
from jax.experimental.pallas import tpu as pltpu
from jax.experimental.pallas import tpu_sc as plsc

N_NODES = 10000
N_EDGES = 320000
N_LG = 640000
N_GRAPHS = 128
D = 128
N_ITER = 3
SND = 6 * D

ROW_BLOCK = 1000


_E = 6400
_CHUNK = 128
_NSUB = 16


def _ptr_pad(n):
    return (n + 15) // 16 * 16


def _vext(vref, idx):
    base = pl.multiple_of((idx >> 4) << 4, 8)
    grp = vref[pl.ds(base, 16)]
    msk = lax.broadcasted_iota(jnp.int32, (16,), 0) == (idx & 15)
    return jnp.sum(jnp.where(msk, grp, 0), axis=0)


_IBLK = 1024
_NCH = _IBLK // _CHUNK


def _zrows(ER):
    for d in range(min(128, ER), 0, -1):
        if ER % d == 0:
            return d


def _seg_body(T, E, init_zero, src_hbm, base_hbm, ssrc_hbm, sdst_hbm, tptr_hbm,
              out_hbm, tptr_v, isrc_v, idst_v, ldstA, ldstB, rowsA, rowsB,
              zbuf, acc_sh, semA, semB):
    c = lax.axis_index("c")
    s = lax.axis_index("s")
    ER = E // _NSUB
    ZR = _zrows(ER)
    pltpu.sync_copy(tptr_hbm, tptr_v)
    if init_zero:
        pltpu.sync_copy(base_hbm, zbuf)
    n_my_tiles = (T - c + 1) // 2
    iota16 = lax.broadcasted_iota(jnp.int32, (16,), 0)

    def tile_body(i, carry):
        t = c + 2 * i
        tbase = t * E
        if init_zero:
            for k in range(ER // ZR):
                pltpu.sync_copy(zbuf, acc_sh.at[pl.ds(s * ER + k * ZR, ZR)])
        else:
            pltpu.sync_copy(base_hbm.at[pl.ds(tbase + s * ER, ER)],
                            acc_sh.at[pl.ds(s * ER, ER)])
        plsc.subcore_barrier()
        a = _vext(tptr_v, t)
        b = _vext(tptr_v, t + 1)
        lo = (a >> 3) << 3
        w8 = ((b + 7) >> 3) - (a >> 3)
        p0 = lo + ((w8 * s) >> 4) * 8
        p1 = lo + ((w8 * (s + 1)) >> 4) * 8
        nblk = (p1 - p0 + _IBLK - 1) // _IBLK

        rows = (rowsA, rowsB)
        sems = (semA, semB)
        lds = (ldstA, ldstB)

        def blk_body(bi, carry2):
            boff = pl.multiple_of(p0 + bi * _IBLK, 8)
            pltpu.sync_copy(ssrc_hbm.at[pl.ds(boff, _IBLK)], isrc_v)
            pltpu.sync_copy(sdst_hbm.at[pl.ds(boff, _IBLK)], idst_v)

            def compute_ldst(ci, lref):
                for v in range(_CHUNK // 16):
                    o = ci * _CHUNK + v * 16
                    dv = idst_v[pl.ds(o, 16)]
                    ld = dv - tbase
                    slot = boff + o + iota16
                    ok = (ld >= 0) & (ld < E) & (slot < p1)
                    lref[pl.ds(v * 16, 16)] = jnp.where(ok, ld, E)

            def start(ci):
                @pl.when(boff + ci * _CHUNK < p1)
                def _():
                    pltpu.async_copy(
                        src_hbm.at[isrc_v.at[pl.ds(ci * _CHUNK, _CHUNK)]],
                        rows[ci % 2], sems[ci % 2])

            def finish(ci):
                @pl.when(boff + ci * _CHUNK < p1)
                def _():
                    lref = lds[ci % 2]
                    compute_ldst(ci, lref)
                    pltpu.make_async_copy(
                        src_hbm.at[isrc_v.at[pl.ds(ci * _CHUNK, _CHUNK)]],
                        rows[ci % 2], sems[ci % 2]).wait()
                    pltpu.sync_copy(rows[ci % 2], acc_sh.at[lref], add=True)

            for ci in range(_NCH):
                start(ci)
                if ci >= 1:
                    finish(ci - 1)
            finish(_NCH - 1)
            return carry2

        lax.fori_loop(0, nblk, blk_body, 0)
        plsc.subcore_barrier()
        pltpu.sync_copy(acc_sh.at[pl.ds(s * ER, ER)],
                        out_hbm.at[pl.ds(tbase + s * ER, ER)])
        plsc.subcore_barrier()
        return carry

    lax.fori_loop(0, n_my_tiles, tile_body, 0)


def _sc_segsum(src, base, ssrc_pad, sdst_pad, tptr, n_out, E, init_zero=False):
    T = n_out // E
    mesh = plsc.VectorSubcoreMesh(core_axis_name="c", subcore_axis_name="s")
    body = functools.partial(_seg_body, T, E, init_zero)
    cp = pltpu.CompilerParams()
    if "needs_layout_passes" in pltpu.CompilerParams.__dataclass_fields__:
        cp = dataclasses.replace(cp, needs_layout_passes=False)
    f = pl.kernel(
        body,
        out_type=jax.ShapeDtypeStruct((n_out, D), jnp.float32),
        mesh=mesh,
        compiler_params=cp,
        scratch_types=[
            pltpu.VMEM((_ptr_pad(T + 1),), jnp.int32),
            pltpu.VMEM((_IBLK,), jnp.int32),
            pltpu.VMEM((_IBLK,), jnp.int32),
            pltpu.VMEM((_CHUNK,), jnp.int32),
            pltpu.VMEM((_CHUNK,), jnp.int32),
            pltpu.VMEM((_CHUNK, D), jnp.float32),
            pltpu.VMEM((_CHUNK, D), jnp.float32),
            pltpu.VMEM((_zrows(E // _NSUB), D), jnp.float32),
            pltpu.VMEM_SHARED((E + 8, D), jnp.float32),
            pltpu.SemaphoreType.DMA,
            pltpu.SemaphoreType.DMA,
        ],
    )
    return f(src, base, ssrc_pad, sdst_pad, tptr)


def _sort_lg(lg_dst, lg_src, n_out, E):
    sdst, ssrc = lax.sort((lg_dst, lg_src), dimension=0, num_keys=1)
    T = n_out // E
    tptr = jnp.searchsorted(sdst, jnp.arange(T + 1, dtype=jnp.int32) * E).astype(jnp.int32)
    tptr = jnp.concatenate([tptr, jnp.full((_ptr_pad(T + 1) - (T + 1),), sdst.shape[0], jnp.int32)])
    ssrc_pad = jnp.concatenate([ssrc, jnp.zeros((2 * _IBLK,), jnp.int32)])
    sdst_pad = jnp.concatenate([sdst, jnp.full((2 * _IBLK,), n_out, jnp.int32)])
    return ssrc_pad, sdst_pad, tptr


def _stats_body(x_ref, o_ref):
    i = pl.program_id(0)

    @pl.when(i == 0)
    def _():
        o_ref[...] = jnp.zeros_like(o_ref)

    xb = x_ref[...]
    o_ref[0, :] += jnp.sum(xb, axis=0)
    o_ref[1, :] += jnp.sum(xb * xb, axis=0)


def _col_stats(x):
    n, c = x.shape
    return pl.pallas_call(
        _stats_body,
        grid=(n // ROW_BLOCK,),
        in_specs=[pl.BlockSpec((ROW_BLOCK, c), lambda i: (i, 0))],
        out_specs=pl.BlockSpec((2, c), lambda i: (0, 0)),
        out_shape=jax.ShapeDtypeStruct((2, c), jnp.float32),
    )(x)


def _apply_body(x_ref, s_ref, g_ref, b_ref, w_ref, c_ref, p_ref, o_ref, *, n, use_prelu):
    m = s_ref[0, :] / n
    v = s_ref[1, :] / n - m * m
    xn = (x_ref[...] - m[None, :]) * (g_ref[0, :] / jnp.sqrt(v + 1e-5))[None, :] + b_ref[0, :][None, :]
    if use_prelu:
        p = p_ref[0, 0]
        xn = jnp.where(xn >= 0, xn, p * xn)
    o_ref[...] = jnp.dot(xn, w_ref[...], preferred_element_type=jnp.float32) + c_ref[0, :][None, :]


def _bn_prelu_matmul(x, stats, g, b, w, c, p):
    n, cin = x.shape
    cout = w.shape[1]
    use_prelu = p is not None
    if p is None:
        p_arr = jnp.zeros((1, 1), jnp.float32)
    else:
        p_arr = jnp.asarray(p, jnp.float32).reshape(1, 1)
    body = functools.partial(_apply_body, n=float(n), use_prelu=use_prelu)
    return pl.pallas_call(
        body,
        grid=(n // ROW_BLOCK,),
        in_specs=[
            pl.BlockSpec((ROW_BLOCK, cin), lambda i: (i, 0)),
            pl.BlockSpec((2, cin), lambda i: (0, 0)),
            pl.BlockSpec((1, cin), lambda i: (0, 0)),
            pl.BlockSpec((1, cin), lambda i: (0, 0)),
            pl.BlockSpec((cin, cout), lambda i: (0, 0)),
            pl.BlockSpec((1, cout), lambda i: (0, 0)),
            pl.BlockSpec((1, 1), lambda i: (0, 0)),
        ],
        out_specs=pl.BlockSpec((ROW_BLOCK, cout), lambda i: (i, 0)),
        out_shape=jax.ShapeDtypeStruct((n, cout), jnp.float32),
    )(x, stats, g.reshape(1, -1), b.reshape(1, -1), w, c.reshape(1, -1), p_arr)


_PBLK = 2000


def _smax_body(xc_ref, b_ref, o_ref):
    i = pl.program_id(0)

    @pl.when(i == 0)
    def _():
        o_ref[...] = jnp.full_like(o_ref, -1e30)

    gid = lax.broadcasted_iota(jnp.int32, (1, N_GRAPHS), 1)
    oh = b_ref[...] == gid
    vals = jnp.where(oh, xc_ref[...], -1e30)
    o_ref[...] = jnp.maximum(o_ref[...], jnp.max(vals, axis=0, keepdims=True))


def _pool_body(xc_ref, b_ref, out_ref, smax_ref, den_ref, gx_ref):
    i = pl.program_id(0)

    @pl.when(i == 0)
    def _():
        den_ref[...] = jnp.zeros_like(den_ref)
        gx_ref[...] = jnp.zeros_like(gx_ref)

    gid = lax.broadcasted_iota(jnp.int32, (1, N_GRAPHS), 1)
    oh = b_ref[...] == gid
    smax_sel = jnp.max(jnp.where(oh, smax_ref[...], -1e30), axis=1, keepdims=True)
    ex = jnp.exp(xc_ref[...] - smax_sel)
    exoh = oh.astype(jnp.float32) * ex
    den_ref[...] += jnp.sum(exoh, axis=0, keepdims=True)
    gx_ref[...] += lax.dot_general(exoh, out_ref[...], (((0,), (0,)), ((), ())),
                                   preferred_element_type=jnp.float32)


def _pool(xc, batch_col, out):
    nb = N_EDGES // _PBLK
    colspec = pl.BlockSpec((_PBLK, 1), lambda i: (i, 0))
    smax = pl.pallas_call(
        _smax_body,
        grid=(nb,),
        in_specs=[colspec, colspec],
        out_specs=pl.BlockSpec((1, N_GRAPHS), lambda i: (0, 0)),
        out_shape=jax.ShapeDtypeStruct((1, N_GRAPHS), jnp.float32),
    )(xc, batch_col)
    den, gxr = pl.pallas_call(
        _pool_body,
        grid=(nb,),
        in_specs=[
            colspec,
            colspec,
            pl.BlockSpec((_PBLK, D), lambda i: (i, 0)),
            pl.BlockSpec((1, N_GRAPHS), lambda i: (0, 0)),
        ],
        out_specs=[
            pl.BlockSpec((1, N_GRAPHS), lambda i: (0, 0)),
            pl.BlockSpec((N_GRAPHS, D), lambda i: (0, 0)),
        ],
        out_shape=[
            jax.ShapeDtypeStruct((1, N_GRAPHS), jnp.float32),
            jax.ShapeDtypeStruct((N_GRAPHS, D), jnp.float32),
        ],
    )(xc, batch_col, out, smax)
    return gxr / jnp.maximum(den, 1e-30).T


def _comb_body(o1_ref, o2_ref, o3_ref, b_ref, w1_ref, w2_ref, w3_ref, out_ref):
    gid = lax.broadcasted_iota(jnp.int32, (1, N_GRAPHS), 1)
    oh = (b_ref[...] == gid).astype(jnp.float32)
    dn = (((1,), (0,)), ((), ()))
    w1 = lax.dot_general(oh, w1_ref[...], dn, preferred_element_type=jnp.float32)
    w2 = lax.dot_general(oh, w2_ref[...], dn, preferred_element_type=jnp.float32)
    w3 = lax.dot_general(oh, w3_ref[...], dn, preferred_element_type=jnp.float32)
    out_ref[...] = o1_ref[...] * w1 + o2_ref[...] * w2 + o3_ref[...] * w3


def _combine(outs, batch_col, scg):
    nb = N_EDGES // _PBLK
    rowspec = pl.BlockSpec((_PBLK, D), lambda i: (i, 0))
    colspec = pl.BlockSpec((_PBLK, 1), lambda i: (i, 0))
    wspec = pl.BlockSpec((N_GRAPHS, 1), lambda i: (0, 0))
    return pl.pallas_call(
        _comb_body,
        grid=(nb,),
        in_specs=[rowspec, rowspec, rowspec, colspec, wspec, wspec, wspec],
        out_specs=rowspec,
        out_shape=jax.ShapeDtypeStruct((N_EDGES, D), jnp.float32),
    )(outs[0], outs[1], outs[2], batch_col,
      scg[:, 0:1], scg[:, 1:2], scg[:, 2:3])


_NODE_PAD = 10240


def _nodeagg_body(src_hbm, ei1_hbm, zeros_hbm, out_hbm,
                  idxA, idxB, rowsA, rowsB, zbuf, acc_sh, semA, semB):
    c = lax.axis_index("c")
    s = lax.axis_index("s")
    ER = _NODE_PAD // _NSUB
    ZR = 64
    idx = (idxA, idxB)
    rows = (rowsA, rowsB)
    sems = (semA, semB)
    pltpu.sync_copy(zeros_hbm, zbuf)
    for k in range(ER // ZR):
        pltpu.sync_copy(zbuf, acc_sh.at[pl.ds(s * ER + k * ZR, ZR)])
    plsc.subcore_barrier()

    def chunk_io(j, buf):
        ci = c + 2 * (s + _NSUB * j)
        off = pl.multiple_of(ci * _CHUNK, 8)
        pltpu.sync_copy(ei1_hbm.at[pl.ds(off, _CHUNK)], idx[buf])
        return pltpu.async_copy(src_hbm.at[pl.ds(off, _CHUNK)], rows[buf], sems[buf])

    def finishc(cp, buf):
        cp.wait()
        pltpu.sync_copy(rows[buf], acc_sh.at[idx[buf]], add=True)

    n_pairs = (N_EDGES // _CHUNK) // (2 * _NSUB * 2)

    def pair(k, carry):
        cp0 = chunk_io(2 * k, 0)
        cp1 = chunk_io(2 * k + 1, 1)
        finishc(cp0, 0)
        finishc(cp1, 1)
        return carry

    lax.fori_loop(0, n_pairs, pair, 0)

    @pl.when(s < 2)
    def _():
        finishc(chunk_io(2 * n_pairs, 0), 0)

    plsc.subcore_barrier()
    pltpu.sync_copy(acc_sh.at[pl.ds(s * ER, ER)],
                    out_hbm.at[c].at[pl.ds(s * ER, ER)])


def _sc_nodeagg(src, ei1):
    mesh = plsc.VectorSubcoreMesh(core_axis_name="c", subcore_axis_name="s")
    cp = pltpu.CompilerParams()
    if "needs_layout_passes" in pltpu.CompilerParams.__dataclass_fields__:
        cp = dataclasses.replace(cp, needs_layout_passes=False)
    f = pl.kernel(
        _nodeagg_body,
        out_type=jax.ShapeDtypeStruct((2, _NODE_PAD, D), jnp.float32),
        mesh=mesh,
        compiler_params=cp,
        scratch_types=[
            pltpu.VMEM((_CHUNK,), jnp.int32),
            pltpu.VMEM((_CHUNK,), jnp.int32),
            pltpu.VMEM((_CHUNK, D), jnp.float32),
            pltpu.VMEM((_CHUNK, D), jnp.float32),
            pltpu.VMEM((64, D), jnp.float32),
            pltpu.VMEM_SHARED((_NODE_PAD, D), jnp.float32),
            pltpu.SemaphoreType.DMA,
            pltpu.SemaphoreType.DMA,
        ],
    )
    parts = f(src, ei1, jnp.zeros((64, D), jnp.float32))
    return parts[0, :N_NODES] + parts[1, :N_NODES]


def _ea_body(eu_hbm, ev_hbm, euv_hbm, ei0_hbm, ei1_hbm, out_hbm,
             i0A, i0B, i1A, i1B, rA0, rA1, rB0, rB1, rC0, rC1,
             sA0, sA1, sB0, sB1, sC0, sC1, sW0, sW1):
    c = lax.axis_index("c")
    s = lax.axis_index("s")
    i0 = (i0A, i0B)
    i1 = (i1A, i1B)
    rA = (rA0, rA1)
    rB = (rB0, rB1)
    rC = (rC0, rC1)
    sA = (sA0, sA1)
    sB = (sB0, sB1)
    sC = (sC0, sC1)
    sW = (sW0, sW1)

    def chunk_io(j, h):
        ci = c + 2 * (s + _NSUB * j)
        off = pl.multiple_of(ci * _CHUNK, 8)
        pltpu.sync_copy(ei0_hbm.at[pl.ds(off, _CHUNK)], i0[h])
        pltpu.sync_copy(ei1_hbm.at[pl.ds(off, _CHUNK)], i1[h])
        cpa = pltpu.async_copy(eu_hbm.at[i0[h]], rA[h], sA[h])
        cpb = pltpu.async_copy(ev_hbm.at[i1[h]], rB[h], sB[h])
        cpc = pltpu.async_copy(euv_hbm.at[pl.ds(off, _CHUNK)], rC[h], sC[h])
        return (cpa, cpb, cpc, off)

    def finishc(cps, h):
        cpa, cpb, cpc, off = cps
        cpa.wait()
        cpb.wait()
        cpc.wait()

        def row(r, carry):
            for g in range(D // 16):
                sl = pl.ds(g * 16, 16)
                rC[h][r, sl] = (rA[h][r, sl] + rB[h][r, sl] + rC[h][r, sl]) * (1.0 / 3.0)
            return carry

        lax.fori_loop(0, _CHUNK, row, 0)
        return pltpu.async_copy(rC[h], out_hbm.at[pl.ds(off, _CHUNK)], sW[h])

    n_pairs = (N_EDGES // _CHUNK) // (2 * _NSUB * 2)
    wr = [None, None]

    def pair(k, carry):
        cps0 = chunk_io(2 * k, 0)
        cps1 = chunk_io(2 * k + 1, 1)
        w0 = finishc(cps0, 0)
        w1 = finishc(cps1, 1)
        w0.wait()
        w1.wait()
        return carry

    lax.fori_loop(0, n_pairs, pair, 0)

    @pl.when(s < 2)
    def _():
        finishc(chunk_io(2 * n_pairs, 0), 0).wait()


def _sc_ea(eu, ev, euv, ei0, ei1):
    mesh = plsc.VectorSubcoreMesh(core_axis_name="c", subcore_axis_name="s")
    cp = pltpu.CompilerParams()
    if "needs_layout_passes" in pltpu.CompilerParams.__dataclass_fields__:
        cp = dataclasses.replace(cp, needs_layout_passes=False)
    f = pl.kernel(
        _ea_body,
        out_type=jax.ShapeDtypeStruct((N_EDGES, D), jnp.float32),
        mesh=mesh,
        compiler_params=cp,
        scratch_types=(
            [pltpu.VMEM((_CHUNK,), jnp.int32)] * 4
            + [pltpu.VMEM((_CHUNK, D), jnp.float32)] * 6
            + [pltpu.SemaphoreType.DMA] * 8
        ),
    )
    return f(eu, ev, euv, ei0, ei1)


def _stage(x, p, idx, use_prelu):
    i = str(idx)
    stats = _col_stats(x)
    pr = p['p' + i] if use_prelu else None
    return _bn_prelu_matmul(x, stats, p['g' + i], p['b' + i], p['W' + i], p['c' + i], pr)


def _linear_block_pallas(x, p):
    x1 = _stage(x, p, 1, False)
    x2 = _stage(x1, p, 2, True)
    x3 = _stage(x2, p, 3, True)
    xm = (x3 + x1) / 2.0
    x4 = _stage(xm, p, 4, True)
    xm2 = (x4 + xm) / 2.0
    return _stage(xm2, p, 5, True)


def kernel(x, edge_attr, params, edge_index, line_graph_edge_index, edge_index_batch):
    lg = line_graph_edge_index
    batch = edge_index_batch
    eu = x @ params['Wu']
    ev = x @ params['Wv']
    euv = edge_attr @ params['We']
    ea = _sc_ea(eu, ev, euv, edge_index[0], edge_index[1])
    ssrc_pad, sdst_pad, tptr = _sort_lg(lg[1], lg[0], N_EDGES, _E)
    zeros_small = jnp.zeros((_zrows(_E // _NSUB), D), jnp.float32)
    batch_col = batch.reshape(-1, 1)
    out = ea
    out_list = []
    gout_list = []
    for _ in range(N_ITER):
        out = _sc_segsum(out, ea, ssrc_pad, sdst_pad, tptr, N_EDGES, _E)
        conv_agg = _sc_segsum(out, zeros_small, ssrc_pad, sdst_pad, tptr, N_EDGES, _E,
                              init_zero=True)
        xc = conv_agg @ params['Wrel'] + params['crel'] + out @ params['Wroot']
        gx = _pool(xc, batch_col, out)
        out_list.append(out)
        gout_list.append(jnp.tanh(gx @ params['Wgout'] + params['cgout']))
    gout_all = jnp.stack(gout_list, axis=-1)
    sc = jnp.sum(gout_all * params['a'], axis=1, keepdims=True) + params['a_bias']
    sc = jax.nn.softmax(sc, axis=-1)
    scg = sc[:, 0, :]
    out = _combine(out_list, batch_col, scg)
    node_agg = _sc_nodeagg(out, edge_index[1])
    h = x + node_agg
    return _linear_block_pallas(h, params)
